# Initial kernel scaffold; baseline (speedup 1.0000x reference)
import jax
import jax.numpy as jnp
from jax.experimental import pallas as pl


def kernel(pos, edge_index, vector_features):
    raise NotImplementedError("write your pallas kernel here")


# speedup vs baseline: 8.7402x; 8.7402x over previous
import functools

import jax
import jax.numpy as jnp
from jax import lax
from jax.experimental import pallas as pl
from jax.experimental.pallas import tpu as pltpu
from jax.experimental.pallas import tpu_sc as plsc

N_NODES = 10000
N_EDGES = 320000
HIDDEN = 128

NPAD = 10240
L = 16
NC = 2
NS = 16
NW = NC * NS
B = 80
ROWS = N_EDGES // B
RPT = ROWS // NW
CROWS = 25
NCHUNK = RPT // CROWS
NSL = NPAD // NS

_mesh = plsc.VectorSubcoreMesh(core_axis_name="c", subcore_axis_name="s")
_sc_params = pltpu.CompilerParams(use_tc_tiling_on_sc=False, needs_layout_passes=False)


def _rsqrt(s):
  i = plsc.bitcast(s, jnp.int32)
  i = jnp.int32(0x5F3759DF) - lax.shift_right_logical(i, 1)
  y = plsc.bitcast(i, jnp.float32)
  hs = s * jnp.float32(0.5)
  for _ in range(3):
    y = y * (jnp.float32(1.5) - hs * y * y)
  return y


@functools.partial(
    pl.kernel,
    out_type=(
        jax.ShapeDtypeStruct((3, ROWS, B), jnp.float32),
        jax.ShapeDtypeStruct((NC, 3, NPAD), jnp.float32),
    ),
    mesh=_mesh,
    compiler_params=_sc_params,
    scratch_types=dict(
        px=pltpu.VMEM((NPAD,), jnp.float32),
        py=pltpu.VMEM((NPAD,), jnp.float32),
        pz=pltpu.VMEM((NPAD,), jnp.float32),
        ridx=pltpu.VMEM((CROWS, B), jnp.int32),
        cidx=pltpu.VMEM((CROWS, B), jnp.int32),
        ux=pltpu.VMEM((CROWS, B), jnp.float32),
        uy=pltpu.VMEM((CROWS, B), jnp.float32),
        uz=pltpu.VMEM((CROWS, B), jnp.float32),
        nux=pltpu.VMEM((CROWS, B), jnp.float32),
        nuy=pltpu.VMEM((CROWS, B), jnp.float32),
        nuz=pltpu.VMEM((CROWS, B), jnp.float32),
        zb=pltpu.VMEM((NSL,), jnp.float32),
        accx=pltpu.VMEM_SHARED((NPAD,), jnp.float32),
        accy=pltpu.VMEM_SHARED((NPAD,), jnp.float32),
        accz=pltpu.VMEM_SHARED((NPAD,), jnp.float32),
    ),
)
def _sc_pass_a(pos_hbm, row_hbm, col_hbm, u_hbm, du_hbm, *, px, py, pz,
               ridx, cidx, ux, uy, uz, nux, nuy, nuz, zb, accx, accy, accz):
  cid = lax.axis_index("c")
  sid = lax.axis_index("s")
  wid = sid * NC + cid

  pltpu.sync_copy(pos_hbm.at[0], px)
  pltpu.sync_copy(pos_hbm.at[1], py)
  pltpu.sync_copy(pos_hbm.at[2], pz)

  for i in range(NSL // L):
    zb[pl.ds(i * L, L)] = jnp.zeros((L,), jnp.float32)
  pltpu.sync_copy(zb, accx.at[pl.ds(sid * NSL, NSL)])
  pltpu.sync_copy(zb, accy.at[pl.ds(sid * NSL, NSL)])
  pltpu.sync_copy(zb, accz.at[pl.ds(sid * NSL, NSL)])
  plsc.subcore_barrier()

  for ch in range(NCHUNK):
    rbase = wid * RPT + ch * CROWS
    pltpu.sync_copy(row_hbm.at[pl.ds(rbase, CROWS)], ridx)
    pltpu.sync_copy(col_hbm.at[pl.ds(rbase, CROWS)], cidx)

    @pl.loop(0, CROWS)
    def _row_loop(j):
      for o in range(B // L):
        sl = pl.ds(o * L, L)
        r16 = ridx[j, sl]
        c16 = cidx[j, sl]
        ex = plsc.load_gather(px, [c16]) - plsc.load_gather(px, [r16])
        ey = plsc.load_gather(py, [c16]) - plsc.load_gather(py, [r16])
        ez = plsc.load_gather(pz, [c16]) - plsc.load_gather(pz, [r16])
        s = ex * ex + ey * ey + ez * ez
        norm = s * _rsqrt(s)
        inv = jnp.float32(1.0) / (norm + jnp.float32(1e-8))
        vx = ex * inv
        vy = ey * inv
        vz = ez * inv
        ux[j, sl] = vx
        uy[j, sl] = vy
        uz[j, sl] = vz
        nux[j, sl] = -vx
        nuy[j, sl] = -vy
        nuz[j, sl] = -vz

    pltpu.sync_copy(ux, u_hbm.at[0].at[pl.ds(rbase, CROWS)])
    pltpu.sync_copy(uy, u_hbm.at[1].at[pl.ds(rbase, CROWS)])
    pltpu.sync_copy(uz, u_hbm.at[2].at[pl.ds(rbase, CROWS)])
    @pl.loop(0, CROWS)
    def _scatter_loop(j):
      pltpu.sync_copy(ux.at[j], accx.at[ridx.at[j]], add=True)
      pltpu.sync_copy(uy.at[j], accy.at[ridx.at[j]], add=True)
      pltpu.sync_copy(uz.at[j], accz.at[ridx.at[j]], add=True)
      pltpu.sync_copy(nux.at[j], accx.at[cidx.at[j]], add=True)
      pltpu.sync_copy(nuy.at[j], accy.at[cidx.at[j]], add=True)
      pltpu.sync_copy(nuz.at[j], accz.at[cidx.at[j]], add=True)

  plsc.subcore_barrier()
  nsl = pl.ds(sid * NSL, NSL)
  pltpu.sync_copy(accx.at[nsl], du_hbm.at[cid].at[0].at[nsl])
  pltpu.sync_copy(accy.at[nsl], du_hbm.at[cid].at[1].at[nsl])
  pltpu.sync_copy(accz.at[nsl], du_hbm.at[cid].at[2].at[nsl])


@functools.partial(
    pl.kernel,
    out_type=(
        jax.ShapeDtypeStruct((ROWS, B), jnp.float32),
        jax.ShapeDtypeStruct((NPAD,), jnp.float32),
        jax.ShapeDtypeStruct((3, NPAD), jnp.float32),
    ),
    mesh=_mesh,
    compiler_params=_sc_params,
    scratch_types=dict(
        dx=pltpu.VMEM((NPAD,), jnp.float32),
        dy=pltpu.VMEM((NPAD,), jnp.float32),
        dz=pltpu.VMEM((NPAD,), jnp.float32),
        tmp=pltpu.VMEM((NPAD,), jnp.float32),
        ridx=pltpu.VMEM((CROWS, B), jnp.int32),
        cidx=pltpu.VMEM((CROWS, B), jnp.int32),
        uxv=pltpu.VMEM((CROWS, B), jnp.float32),
        uyv=pltpu.VMEM((CROWS, B), jnp.float32),
        uzv=pltpu.VMEM((CROWS, B), jnp.float32),
        dih=pltpu.VMEM((CROWS, B), jnp.float32),
        angv=pltpu.VMEM((NSL,), jnp.float32),
    ),
)
def _sc_pass_b(row_hbm, col_hbm, u_hbm, du_hbm, dih_hbm, ang_hbm, dsum_hbm, *,
               dx, dy, dz, tmp, ridx, cidx, uxv, uyv, uzv, dih, angv):
  cid = lax.axis_index("c")
  sid = lax.axis_index("s")
  wid = sid * NC + cid

  for comp, d in ((0, dx), (1, dy), (2, dz)):
    pltpu.sync_copy(du_hbm.at[0].at[comp], d)
    pltpu.sync_copy(du_hbm.at[1].at[comp], tmp)

    @pl.loop(0, NPAD // L)
    def _add_loop(i):
      sl = pl.ds(i * L, L)
      d[sl] = d[sl] + tmp[sl]

  @pl.when(cid == 0)
  def _emit_nodes():
    base = sid * NSL

    @pl.loop(0, NSL // L)
    def _ang_loop(i):
      sl = pl.ds(i * L, L)
      vdx = dx[pl.ds(base + i * L, L)]
      vdy = dy[pl.ds(base + i * L, L)]
      vdz = dz[pl.ds(base + i * L, L)]
      angv[sl] = vdx * vdx + vdy * vdy + vdz * vdz

    nsl = pl.ds(base, NSL)
    pltpu.sync_copy(angv, ang_hbm.at[nsl])
    pltpu.sync_copy(dx.at[nsl], dsum_hbm.at[0].at[nsl])
    pltpu.sync_copy(dy.at[nsl], dsum_hbm.at[1].at[nsl])
    pltpu.sync_copy(dz.at[nsl], dsum_hbm.at[2].at[nsl])

  for ch in range(NCHUNK):
    rbase = wid * RPT + ch * CROWS
    pltpu.sync_copy(row_hbm.at[pl.ds(rbase, CROWS)], ridx)
    pltpu.sync_copy(col_hbm.at[pl.ds(rbase, CROWS)], cidx)
    pltpu.sync_copy(u_hbm.at[0].at[pl.ds(rbase, CROWS)], uxv)
    pltpu.sync_copy(u_hbm.at[1].at[pl.ds(rbase, CROWS)], uyv)
    pltpu.sync_copy(u_hbm.at[2].at[pl.ds(rbase, CROWS)], uzv)

    @pl.loop(0, CROWS)
    def _row_loop(j):
      for o in range(B // L):
        sl = pl.ds(o * L, L)
        r16 = ridx[j, sl]
        c16 = cidx[j, sl]
        kx = uxv[j, sl]
        ky = uyv[j, sl]
        kz = uzv[j, sl]
        vix = plsc.load_gather(dx, [r16])
        viy = plsc.load_gather(dy, [r16])
        viz = plsc.load_gather(dz, [r16])
        vjx = plsc.load_gather(dx, [c16])
        vjy = plsc.load_gather(dy, [c16])
        vjz = plsc.load_gather(dz, [c16])
        a = vix * kx + viy * ky + viz * kz
        b = vjx * kx + vjy * ky + vjz * kz
        ss = kx * kx + ky * ky + kz * kz
        dot = vix * vjx + viy * vjy + viz * vjz
        dih[j, sl] = dot - a * b * (jnp.float32(2.0) - ss)

    pltpu.sync_copy(dih, dih_hbm.at[pl.ds(rbase, CROWS)])


def _tc_broadcast(x, nrows, blk):
  def body(x_ref, o_ref):
    o_ref[...] = jnp.broadcast_to(x_ref[...], (blk, HIDDEN))

  return pl.pallas_call(
      body,
      grid=(nrows // blk,),
      in_specs=[pl.BlockSpec((blk, 1), lambda i: (i, 0))],
      out_specs=pl.BlockSpec((blk, HIDDEN), lambda i: (i, 0)),
      out_shape=jax.ShapeDtypeStruct((nrows, HIDDEN), jnp.float32),
  )(x)


@jax.jit
def kernel(pos, edge_index, vector_features):
  del vector_features
  pos_t = jnp.zeros((3, NPAD), jnp.float32).at[:, :N_NODES].set(pos.T)
  row2d = edge_index[0].reshape(ROWS, B)
  col2d = edge_index[1].reshape(ROWS, B)

  u_hbm, du_part = _sc_pass_a(pos_t, row2d, col2d)
  dih, ang, dsum = _sc_pass_b(row2d, col2d, u_hbm, du_part)

  dihedral_info = _tc_broadcast(dih.reshape(N_EDGES, 1), N_EDGES, 1000)
  angular_pad = _tc_broadcast(ang.reshape(NPAD, 1), NPAD, 640)

  angular_info = angular_pad[:N_NODES]
  direction_units = dsum[:, :N_NODES].T
  return (angular_info, dihedral_info, direction_units)


# speedup vs baseline: 13.5529x; 1.5506x over previous
import functools

import jax
import jax.numpy as jnp
from jax import lax
from jax.experimental import pallas as pl
from jax.experimental.pallas import tpu as pltpu
from jax.experimental.pallas import tpu_sc as plsc

N_NODES = 10000
N_EDGES = 320000
HIDDEN = 128

NPAD = 10240
L = 16
NC = 2
NS = 16
NW = NC * NS
B = 80
ROWS = N_EDGES // B
RPT = ROWS // NW
CROWS = 25
NCHUNK = RPT // CROWS
NSL = NPAD // NS

_mesh = plsc.VectorSubcoreMesh(core_axis_name="c", subcore_axis_name="s")
_sc_params = pltpu.CompilerParams(use_tc_tiling_on_sc=False, needs_layout_passes=False)


def _rsqrt(s):
  i = plsc.bitcast(s, jnp.int32)
  i = jnp.int32(0x5F3759DF) - lax.shift_right_logical(i, 1)
  y = plsc.bitcast(i, jnp.float32)
  hs = s * jnp.float32(0.5)
  for _ in range(3):
    y = y * (jnp.float32(1.5) - hs * y * y)
  return y


@functools.partial(
    pl.kernel,
    out_type=(
        jax.ShapeDtypeStruct((3, ROWS, B), jnp.float32),
        jax.ShapeDtypeStruct((NC, 3, NPAD), jnp.float32),
    ),
    mesh=_mesh,
    compiler_params=_sc_params,
    scratch_types=dict(
        px=pltpu.VMEM((NPAD,), jnp.float32),
        py=pltpu.VMEM((NPAD,), jnp.float32),
        pz=pltpu.VMEM((NPAD,), jnp.float32),
        ridx=pltpu.VMEM((CROWS, B), jnp.int32),
        cidx=pltpu.VMEM((CROWS, B), jnp.int32),
        ux=pltpu.VMEM((CROWS, B), jnp.float32),
        uy=pltpu.VMEM((CROWS, B), jnp.float32),
        uz=pltpu.VMEM((CROWS, B), jnp.float32),
        nux=pltpu.VMEM((CROWS, B), jnp.float32),
        nuy=pltpu.VMEM((CROWS, B), jnp.float32),
        nuz=pltpu.VMEM((CROWS, B), jnp.float32),
        zb=pltpu.VMEM((NSL,), jnp.float32),
        accx=pltpu.VMEM_SHARED((NPAD,), jnp.float32),
        accy=pltpu.VMEM_SHARED((NPAD,), jnp.float32),
        accz=pltpu.VMEM_SHARED((NPAD,), jnp.float32),
    ),
)
def _sc_pass_a(pos_hbm, row_hbm, col_hbm, u_hbm, du_hbm, *, px, py, pz,
               ridx, cidx, ux, uy, uz, nux, nuy, nuz, zb, accx, accy, accz):
  cid = lax.axis_index("c")
  sid = lax.axis_index("s")
  wid = sid * NC + cid

  pltpu.sync_copy(pos_hbm.at[0], px)
  pltpu.sync_copy(pos_hbm.at[1], py)
  pltpu.sync_copy(pos_hbm.at[2], pz)

  for i in range(NSL // L):
    zb[pl.ds(i * L, L)] = jnp.zeros((L,), jnp.float32)
  pltpu.sync_copy(zb, accx.at[pl.ds(sid * NSL, NSL)])
  pltpu.sync_copy(zb, accy.at[pl.ds(sid * NSL, NSL)])
  pltpu.sync_copy(zb, accz.at[pl.ds(sid * NSL, NSL)])
  plsc.subcore_barrier()

  for ch in range(NCHUNK):
    rbase = wid * RPT + ch * CROWS
    pltpu.sync_copy(row_hbm.at[pl.ds(rbase, CROWS)], ridx)
    pltpu.sync_copy(col_hbm.at[pl.ds(rbase, CROWS)], cidx)

    @pl.loop(0, CROWS)
    def _row_loop(j):
      for o in range(B // L):
        sl = pl.ds(o * L, L)
        r16 = ridx[j, sl]
        c16 = cidx[j, sl]
        ex = plsc.load_gather(px, [c16]) - plsc.load_gather(px, [r16])
        ey = plsc.load_gather(py, [c16]) - plsc.load_gather(py, [r16])
        ez = plsc.load_gather(pz, [c16]) - plsc.load_gather(pz, [r16])
        s = ex * ex + ey * ey + ez * ez
        norm = s * _rsqrt(s)
        inv = jnp.float32(1.0) / (norm + jnp.float32(1e-8))
        vx = ex * inv
        vy = ey * inv
        vz = ez * inv
        ux[j, sl] = vx
        uy[j, sl] = vy
        uz[j, sl] = vz
        nux[j, sl] = -vx
        nuy[j, sl] = -vy
        nuz[j, sl] = -vz

    pltpu.sync_copy(ux, u_hbm.at[0].at[pl.ds(rbase, CROWS)])
    pltpu.sync_copy(uy, u_hbm.at[1].at[pl.ds(rbase, CROWS)])
    pltpu.sync_copy(uz, u_hbm.at[2].at[pl.ds(rbase, CROWS)])
    @pl.loop(0, CROWS)
    def _scatter_loop(j):
      pltpu.sync_copy(ux.at[j], accx.at[ridx.at[j]], add=True)
      pltpu.sync_copy(uy.at[j], accy.at[ridx.at[j]], add=True)
      pltpu.sync_copy(uz.at[j], accz.at[ridx.at[j]], add=True)
      pltpu.sync_copy(nux.at[j], accx.at[cidx.at[j]], add=True)
      pltpu.sync_copy(nuy.at[j], accy.at[cidx.at[j]], add=True)
      pltpu.sync_copy(nuz.at[j], accz.at[cidx.at[j]], add=True)

  plsc.subcore_barrier()
  nsl = pl.ds(sid * NSL, NSL)
  pltpu.sync_copy(accx.at[nsl], du_hbm.at[cid].at[0].at[nsl])
  pltpu.sync_copy(accy.at[nsl], du_hbm.at[cid].at[1].at[nsl])
  pltpu.sync_copy(accz.at[nsl], du_hbm.at[cid].at[2].at[nsl])


@functools.partial(
    pl.kernel,
    out_type=(
        jax.ShapeDtypeStruct((ROWS, B), jnp.float32),
        jax.ShapeDtypeStruct((NPAD,), jnp.float32),
        jax.ShapeDtypeStruct((3, NPAD), jnp.float32),
    ),
    mesh=_mesh,
    compiler_params=_sc_params,
    scratch_types=dict(
        dx=pltpu.VMEM((NPAD,), jnp.float32),
        dy=pltpu.VMEM((NPAD,), jnp.float32),
        dz=pltpu.VMEM((NPAD,), jnp.float32),
        tmp=pltpu.VMEM((NPAD,), jnp.float32),
        ridx=pltpu.VMEM((CROWS, B), jnp.int32),
        cidx=pltpu.VMEM((CROWS, B), jnp.int32),
        uxv=pltpu.VMEM((CROWS, B), jnp.float32),
        uyv=pltpu.VMEM((CROWS, B), jnp.float32),
        uzv=pltpu.VMEM((CROWS, B), jnp.float32),
        dih=pltpu.VMEM((CROWS, B), jnp.float32),
        angv=pltpu.VMEM((NSL,), jnp.float32),
    ),
)
def _sc_pass_b(row_hbm, col_hbm, u_hbm, du_hbm, dih_hbm, ang_hbm, dsum_hbm, *,
               dx, dy, dz, tmp, ridx, cidx, uxv, uyv, uzv, dih, angv):
  cid = lax.axis_index("c")
  sid = lax.axis_index("s")
  wid = sid * NC + cid

  for comp, d in ((0, dx), (1, dy), (2, dz)):
    pltpu.sync_copy(du_hbm.at[0].at[comp], d)
    pltpu.sync_copy(du_hbm.at[1].at[comp], tmp)

    @pl.loop(0, NPAD // L)
    def _add_loop(i):
      sl = pl.ds(i * L, L)
      d[sl] = d[sl] + tmp[sl]

  @pl.when(cid == 0)
  def _emit_nodes():
    base = sid * NSL

    @pl.loop(0, NSL // L)
    def _ang_loop(i):
      sl = pl.ds(i * L, L)
      vdx = dx[pl.ds(base + i * L, L)]
      vdy = dy[pl.ds(base + i * L, L)]
      vdz = dz[pl.ds(base + i * L, L)]
      angv[sl] = vdx * vdx + vdy * vdy + vdz * vdz

    nsl = pl.ds(base, NSL)
    pltpu.sync_copy(angv, ang_hbm.at[nsl])
    pltpu.sync_copy(dx.at[nsl], dsum_hbm.at[0].at[nsl])
    pltpu.sync_copy(dy.at[nsl], dsum_hbm.at[1].at[nsl])
    pltpu.sync_copy(dz.at[nsl], dsum_hbm.at[2].at[nsl])

  for ch in range(NCHUNK):
    rbase = wid * RPT + ch * CROWS
    pltpu.sync_copy(row_hbm.at[pl.ds(rbase, CROWS)], ridx)
    pltpu.sync_copy(col_hbm.at[pl.ds(rbase, CROWS)], cidx)
    pltpu.sync_copy(u_hbm.at[0].at[pl.ds(rbase, CROWS)], uxv)
    pltpu.sync_copy(u_hbm.at[1].at[pl.ds(rbase, CROWS)], uyv)
    pltpu.sync_copy(u_hbm.at[2].at[pl.ds(rbase, CROWS)], uzv)

    @pl.loop(0, CROWS)
    def _row_loop(j):
      for o in range(B // L):
        sl = pl.ds(o * L, L)
        r16 = ridx[j, sl]
        c16 = cidx[j, sl]
        kx = uxv[j, sl]
        ky = uyv[j, sl]
        kz = uzv[j, sl]
        vix = plsc.load_gather(dx, [r16])
        viy = plsc.load_gather(dy, [r16])
        viz = plsc.load_gather(dz, [r16])
        vjx = plsc.load_gather(dx, [c16])
        vjy = plsc.load_gather(dy, [c16])
        vjz = plsc.load_gather(dz, [c16])
        a = vix * kx + viy * ky + viz * kz
        b = vjx * kx + vjy * ky + vjz * kz
        ss = kx * kx + ky * ky + kz * kz
        dot = vix * vjx + viy * vjy + viz * vjz
        dih[j, sl] = dot - a * b * (jnp.float32(2.0) - ss)

    pltpu.sync_copy(dih, dih_hbm.at[pl.ds(rbase, CROWS)])


_BCR = 10


def _tc_broadcast(xflat, nrows):
  g = nrows // (_BCR * 128)

  def body(x_ref, o_ref):
    x = x_ref[0]
    o_ref[...] = jnp.broadcast_to(
        x[:, :, None], (_BCR, 128, HIDDEN)
    ).reshape(_BCR * 128, HIDDEN)

  return pl.pallas_call(
      body,
      grid=(g,),
      in_specs=[pl.BlockSpec((1, _BCR, 128), lambda i: (i, 0, 0))],
      out_specs=pl.BlockSpec((_BCR * 128, HIDDEN), lambda i: (i, 0)),
      out_shape=jax.ShapeDtypeStruct((nrows, HIDDEN), jnp.float32),
  )(xflat.reshape(g, _BCR, 128))


@jax.jit
def kernel(pos, edge_index, vector_features):
  del vector_features
  pos_t = jnp.zeros((3, NPAD), jnp.float32).at[:, :N_NODES].set(pos.T)
  row2d = edge_index[0].reshape(ROWS, B)
  col2d = edge_index[1].reshape(ROWS, B)

  u_hbm, du_part = _sc_pass_a(pos_t, row2d, col2d)
  dih, ang, dsum = _sc_pass_b(row2d, col2d, u_hbm, du_part)

  dihedral_info = _tc_broadcast(dih.reshape(N_EDGES), N_EDGES)
  angular_pad = _tc_broadcast(ang, NPAD)

  angular_info = angular_pad[:N_NODES]
  direction_units = dsum[:, :N_NODES].T
  return (angular_info, dihedral_info, direction_units)


# speedup vs baseline: 19.1604x; 1.4138x over previous
import functools

import jax
import jax.numpy as jnp
from jax import lax
from jax.experimental import pallas as pl
from jax.experimental.pallas import tpu as pltpu
from jax.experimental.pallas import tpu_sc as plsc

N_NODES = 10000
N_EDGES = 320000
HIDDEN = 128

NPAD = 10240
L = 16
NC = 2
NS = 16
NW = NC * NS
B = 80
ROWS = N_EDGES // B
RPT = ROWS // NW
CROWS = 25
NCHUNK = RPT // CROWS
NSL = NPAD // NS

_mesh = plsc.VectorSubcoreMesh(core_axis_name="c", subcore_axis_name="s")
_sc_params = pltpu.CompilerParams(use_tc_tiling_on_sc=False, needs_layout_passes=False)


def _rsqrt(s):
  i = plsc.bitcast(s, jnp.int32)
  i = jnp.int32(0x5F3759DF) - lax.shift_right_logical(i, 1)
  y = plsc.bitcast(i, jnp.float32)
  hs = s * jnp.float32(0.5)
  for _ in range(3):
    y = y * (jnp.float32(1.5) - hs * y * y)
  return y


@functools.partial(
    pl.kernel,
    out_type=(
        jax.ShapeDtypeStruct((3, ROWS, B), jnp.float32),
        jax.ShapeDtypeStruct((NC, 3, NPAD), jnp.float32),
    ),
    mesh=_mesh,
    compiler_params=_sc_params,
    scratch_types=dict(
        px=pltpu.VMEM((NPAD,), jnp.float32),
        py=pltpu.VMEM((NPAD,), jnp.float32),
        pz=pltpu.VMEM((NPAD,), jnp.float32),
        ridx=pltpu.VMEM((CROWS, B), jnp.int32),
        cidx=pltpu.VMEM((CROWS, B), jnp.int32),
        ux=pltpu.VMEM((CROWS, B), jnp.float32),
        uy=pltpu.VMEM((CROWS, B), jnp.float32),
        uz=pltpu.VMEM((CROWS, B), jnp.float32),
        nux=pltpu.VMEM((CROWS, B), jnp.float32),
        nuy=pltpu.VMEM((CROWS, B), jnp.float32),
        nuz=pltpu.VMEM((CROWS, B), jnp.float32),
        zb=pltpu.VMEM((NSL,), jnp.float32),
        accx=pltpu.VMEM_SHARED((NPAD,), jnp.float32),
        accy=pltpu.VMEM_SHARED((NPAD,), jnp.float32),
        accz=pltpu.VMEM_SHARED((NPAD,), jnp.float32),
    ),
)
def _sc_pass_a(pos_hbm, row_hbm, col_hbm, u_hbm, du_hbm, *, px, py, pz,
               ridx, cidx, ux, uy, uz, nux, nuy, nuz, zb, accx, accy, accz):
  cid = lax.axis_index("c")
  sid = lax.axis_index("s")
  wid = sid * NC + cid

  pltpu.sync_copy(pos_hbm.at[0], px)
  pltpu.sync_copy(pos_hbm.at[1], py)
  pltpu.sync_copy(pos_hbm.at[2], pz)

  for i in range(NSL // L):
    zb[pl.ds(i * L, L)] = jnp.zeros((L,), jnp.float32)
  pltpu.sync_copy(zb, accx.at[pl.ds(sid * NSL, NSL)])
  pltpu.sync_copy(zb, accy.at[pl.ds(sid * NSL, NSL)])
  pltpu.sync_copy(zb, accz.at[pl.ds(sid * NSL, NSL)])
  plsc.subcore_barrier()

  for ch in range(NCHUNK):
    rbase = wid * RPT + ch * CROWS
    pltpu.sync_copy(row_hbm.at[pl.ds(rbase, CROWS)], ridx)
    pltpu.sync_copy(col_hbm.at[pl.ds(rbase, CROWS)], cidx)

    @pl.loop(0, CROWS)
    def _row_loop(j):
      for o in range(B // L):
        sl = pl.ds(o * L, L)
        r16 = ridx[j, sl]
        c16 = cidx[j, sl]
        ex = plsc.load_gather(px, [c16]) - plsc.load_gather(px, [r16])
        ey = plsc.load_gather(py, [c16]) - plsc.load_gather(py, [r16])
        ez = plsc.load_gather(pz, [c16]) - plsc.load_gather(pz, [r16])
        s = ex * ex + ey * ey + ez * ez
        norm = s * _rsqrt(s)
        inv = jnp.float32(1.0) / (norm + jnp.float32(1e-8))
        vx = ex * inv
        vy = ey * inv
        vz = ez * inv
        ux[j, sl] = vx
        uy[j, sl] = vy
        uz[j, sl] = vz
        nux[j, sl] = -vx
        nuy[j, sl] = -vy
        nuz[j, sl] = -vz

    pltpu.sync_copy(ux, u_hbm.at[0].at[pl.ds(rbase, CROWS)])
    pltpu.sync_copy(uy, u_hbm.at[1].at[pl.ds(rbase, CROWS)])
    pltpu.sync_copy(uz, u_hbm.at[2].at[pl.ds(rbase, CROWS)])
    @pl.loop(0, CROWS)
    def _scatter_loop(j):
      pltpu.sync_copy(
          (ux.at[j], uy.at[j], uz.at[j], nux.at[j], nuy.at[j], nuz.at[j]),
          (accx.at[ridx.at[j]], accy.at[ridx.at[j]], accz.at[ridx.at[j]],
           accx.at[cidx.at[j]], accy.at[cidx.at[j]], accz.at[cidx.at[j]]),
          add=True,
      )

  plsc.subcore_barrier()
  nsl = pl.ds(sid * NSL, NSL)
  pltpu.sync_copy(accx.at[nsl], du_hbm.at[cid].at[0].at[nsl])
  pltpu.sync_copy(accy.at[nsl], du_hbm.at[cid].at[1].at[nsl])
  pltpu.sync_copy(accz.at[nsl], du_hbm.at[cid].at[2].at[nsl])


@functools.partial(
    pl.kernel,
    out_type=(
        jax.ShapeDtypeStruct((ROWS, B), jnp.float32),
        jax.ShapeDtypeStruct((NPAD,), jnp.float32),
        jax.ShapeDtypeStruct((3, NPAD), jnp.float32),
    ),
    mesh=_mesh,
    compiler_params=_sc_params,
    scratch_types=dict(
        dx=pltpu.VMEM((NPAD,), jnp.float32),
        dy=pltpu.VMEM((NPAD,), jnp.float32),
        dz=pltpu.VMEM((NPAD,), jnp.float32),
        tmp=pltpu.VMEM((NPAD,), jnp.float32),
        ridx=pltpu.VMEM((CROWS, B), jnp.int32),
        cidx=pltpu.VMEM((CROWS, B), jnp.int32),
        uxv=pltpu.VMEM((CROWS, B), jnp.float32),
        uyv=pltpu.VMEM((CROWS, B), jnp.float32),
        uzv=pltpu.VMEM((CROWS, B), jnp.float32),
        dih=pltpu.VMEM((CROWS, B), jnp.float32),
        angv=pltpu.VMEM((NSL,), jnp.float32),
    ),
)
def _sc_pass_b(row_hbm, col_hbm, u_hbm, du_hbm, dih_hbm, ang_hbm, dsum_hbm, *,
               dx, dy, dz, tmp, ridx, cidx, uxv, uyv, uzv, dih, angv):
  cid = lax.axis_index("c")
  sid = lax.axis_index("s")
  wid = sid * NC + cid

  for comp, d in ((0, dx), (1, dy), (2, dz)):
    pltpu.sync_copy(du_hbm.at[0].at[comp], d)
    pltpu.sync_copy(du_hbm.at[1].at[comp], tmp)

    @pl.loop(0, NPAD // L)
    def _add_loop(i):
      sl = pl.ds(i * L, L)
      d[sl] = d[sl] + tmp[sl]

  @pl.when(cid == 0)
  def _emit_nodes():
    base = sid * NSL

    @pl.loop(0, NSL // L)
    def _ang_loop(i):
      sl = pl.ds(i * L, L)
      vdx = dx[pl.ds(base + i * L, L)]
      vdy = dy[pl.ds(base + i * L, L)]
      vdz = dz[pl.ds(base + i * L, L)]
      angv[sl] = vdx * vdx + vdy * vdy + vdz * vdz

    nsl = pl.ds(base, NSL)
    pltpu.sync_copy(angv, ang_hbm.at[nsl])
    pltpu.sync_copy(dx.at[nsl], dsum_hbm.at[0].at[nsl])
    pltpu.sync_copy(dy.at[nsl], dsum_hbm.at[1].at[nsl])
    pltpu.sync_copy(dz.at[nsl], dsum_hbm.at[2].at[nsl])

  for ch in range(NCHUNK):
    rbase = wid * RPT + ch * CROWS
    pltpu.sync_copy(row_hbm.at[pl.ds(rbase, CROWS)], ridx)
    pltpu.sync_copy(col_hbm.at[pl.ds(rbase, CROWS)], cidx)
    pltpu.sync_copy(u_hbm.at[0].at[pl.ds(rbase, CROWS)], uxv)
    pltpu.sync_copy(u_hbm.at[1].at[pl.ds(rbase, CROWS)], uyv)
    pltpu.sync_copy(u_hbm.at[2].at[pl.ds(rbase, CROWS)], uzv)

    @pl.loop(0, CROWS)
    def _row_loop(j):
      for o in range(B // L):
        sl = pl.ds(o * L, L)
        r16 = ridx[j, sl]
        c16 = cidx[j, sl]
        kx = uxv[j, sl]
        ky = uyv[j, sl]
        kz = uzv[j, sl]
        vix = plsc.load_gather(dx, [r16])
        viy = plsc.load_gather(dy, [r16])
        viz = plsc.load_gather(dz, [r16])
        vjx = plsc.load_gather(dx, [c16])
        vjy = plsc.load_gather(dy, [c16])
        vjz = plsc.load_gather(dz, [c16])
        a = vix * kx + viy * ky + viz * kz
        b = vjx * kx + vjy * ky + vjz * kz
        ss = kx * kx + ky * ky + kz * kz
        dot = vix * vjx + viy * vjy + viz * vjz
        dih[j, sl] = dot - a * b * (jnp.float32(2.0) - ss)

    pltpu.sync_copy(dih, dih_hbm.at[pl.ds(rbase, CROWS)])


def _tc_broadcast(xflat, nrows, bcr):
  g = nrows // (bcr * 128)

  def body(x_ref, o_ref):
    x = x_ref[0]
    o_ref[...] = jnp.broadcast_to(
        x[:, :, None], (bcr, 128, HIDDEN)
    ).reshape(bcr * 128, HIDDEN)

  return pl.pallas_call(
      body,
      grid=(g,),
      in_specs=[pl.BlockSpec((1, bcr, 128), lambda i: (i, 0, 0))],
      out_specs=pl.BlockSpec((bcr * 128, HIDDEN), lambda i: (i, 0)),
      out_shape=jax.ShapeDtypeStruct((nrows, HIDDEN), jnp.float32),
  )(xflat.reshape(g, bcr, 128))


@jax.jit
def kernel(pos, edge_index, vector_features):
  del vector_features
  pos_t = jnp.zeros((3, NPAD), jnp.float32).at[:, :N_NODES].set(pos.T)
  row2d = edge_index[0].reshape(ROWS, B)
  col2d = edge_index[1].reshape(ROWS, B)

  u_hbm, du_part = _sc_pass_a(pos_t, row2d, col2d)
  dih, ang, dsum = _sc_pass_b(row2d, col2d, u_hbm, du_part)

  dihedral_info = _tc_broadcast(dih.reshape(N_EDGES), N_EDGES, 25)
  angular_pad = _tc_broadcast(ang, NPAD, 10)

  angular_info = angular_pad[:N_NODES]
  direction_units = dsum[:, :N_NODES].T
  return (angular_info, dihedral_info, direction_units)


# speedup vs baseline: 22.5793x; 1.1784x over previous
import functools

import jax
import jax.numpy as jnp
from jax import lax
from jax.experimental import pallas as pl
from jax.experimental.pallas import tpu as pltpu
from jax.experimental.pallas import tpu_sc as plsc

N_NODES = 10000
N_EDGES = 320000
HIDDEN = 128

NPAD = 10240
L = 16
NC = 2
NS = 16
NW = NC * NS
B = 80
ROWS = N_EDGES // B
RPT = ROWS // NW
CROWS = 25
NCHUNK = RPT // CROWS
NSL = NPAD // NS

_mesh = plsc.VectorSubcoreMesh(core_axis_name="c", subcore_axis_name="s")
_sc_params = pltpu.CompilerParams(use_tc_tiling_on_sc=False, needs_layout_passes=False)


def _rsqrt(s):
  i = plsc.bitcast(s, jnp.int32)
  i = jnp.int32(0x5F3759DF) - lax.shift_right_logical(i, 1)
  y = plsc.bitcast(i, jnp.float32)
  hs = s * jnp.float32(0.5)
  for _ in range(3):
    y = y * (jnp.float32(1.5) - hs * y * y)
  return y


@functools.partial(
    pl.kernel,
    out_type=(
        jax.ShapeDtypeStruct((3, ROWS, B), jnp.float32),
        jax.ShapeDtypeStruct((NC, 3, NPAD), jnp.float32),
    ),
    mesh=_mesh,
    compiler_params=_sc_params,
    scratch_types=dict(
        px=pltpu.VMEM((NPAD,), jnp.float32),
        py=pltpu.VMEM((NPAD,), jnp.float32),
        pz=pltpu.VMEM((NPAD,), jnp.float32),
        ridx=pltpu.VMEM((CROWS, B), jnp.int32),
        cidx=pltpu.VMEM((CROWS, B), jnp.int32),
        ux=pltpu.VMEM((CROWS, B), jnp.float32),
        uy=pltpu.VMEM((CROWS, B), jnp.float32),
        uz=pltpu.VMEM((CROWS, B), jnp.float32),
        nux=pltpu.VMEM((CROWS, B), jnp.float32),
        nuy=pltpu.VMEM((CROWS, B), jnp.float32),
        nuz=pltpu.VMEM((CROWS, B), jnp.float32),
        zb=pltpu.VMEM((NSL,), jnp.float32),
        accx=pltpu.VMEM_SHARED((NPAD,), jnp.float32),
        accy=pltpu.VMEM_SHARED((NPAD,), jnp.float32),
        accz=pltpu.VMEM_SHARED((NPAD,), jnp.float32),
        sem=pltpu.SemaphoreType.DMA,
    ),
)
def _sc_pass_a(pos_hbm, row_hbm, col_hbm, u_hbm, du_hbm, *, px, py, pz,
               ridx, cidx, ux, uy, uz, nux, nuy, nuz, zb, accx, accy, accz,
               sem):
  cid = lax.axis_index("c")
  sid = lax.axis_index("s")
  wid = sid * NC + cid

  pltpu.sync_copy(pos_hbm.at[0], px)
  pltpu.sync_copy(pos_hbm.at[1], py)
  pltpu.sync_copy(pos_hbm.at[2], pz)

  for i in range(NSL // L):
    zb[pl.ds(i * L, L)] = jnp.zeros((L,), jnp.float32)
  pltpu.sync_copy(zb, accx.at[pl.ds(sid * NSL, NSL)])
  pltpu.sync_copy(zb, accy.at[pl.ds(sid * NSL, NSL)])
  pltpu.sync_copy(zb, accz.at[pl.ds(sid * NSL, NSL)])
  plsc.subcore_barrier()

  def _scatter_row(j):
    pltpu.async_copy(ux.at[j], accx.at[ridx.at[j]], sem, add=True)
    pltpu.async_copy(uy.at[j], accy.at[ridx.at[j]], sem, add=True)
    pltpu.async_copy(uz.at[j], accz.at[ridx.at[j]], sem, add=True)
    pltpu.async_copy(nux.at[j], accx.at[cidx.at[j]], sem, add=True)
    pltpu.async_copy(nuy.at[j], accy.at[cidx.at[j]], sem, add=True)
    pltpu.async_copy(nuz.at[j], accz.at[cidx.at[j]], sem, add=True)

  def _drain_row(j):
    pltpu.make_async_copy(ux.at[j], accx.at[ridx.at[j]], sem).wait()
    pltpu.make_async_copy(uy.at[j], accy.at[ridx.at[j]], sem).wait()
    pltpu.make_async_copy(uz.at[j], accz.at[ridx.at[j]], sem).wait()
    pltpu.make_async_copy(nux.at[j], accx.at[cidx.at[j]], sem).wait()
    pltpu.make_async_copy(nuy.at[j], accy.at[cidx.at[j]], sem).wait()
    pltpu.make_async_copy(nuz.at[j], accz.at[cidx.at[j]], sem).wait()

  LAG = 5

  for ch in range(NCHUNK):
    rbase = wid * RPT + ch * CROWS
    pltpu.sync_copy(
        (row_hbm.at[pl.ds(rbase, CROWS)], col_hbm.at[pl.ds(rbase, CROWS)]),
        (ridx, cidx),
    )

    @pl.loop(0, CROWS)
    def _row_loop(j):
      for o in range(B // L):
        sl = pl.ds(o * L, L)
        r16 = ridx[j, sl]
        c16 = cidx[j, sl]
        ex = plsc.load_gather(px, [c16]) - plsc.load_gather(px, [r16])
        ey = plsc.load_gather(py, [c16]) - plsc.load_gather(py, [r16])
        ez = plsc.load_gather(pz, [c16]) - plsc.load_gather(pz, [r16])
        s = ex * ex + ey * ey + ez * ez
        norm = s * _rsqrt(s)
        inv = jnp.float32(1.0) / (norm + jnp.float32(1e-8))
        vx = ex * inv
        vy = ey * inv
        vz = ez * inv
        ux[j, sl] = vx
        uy[j, sl] = vy
        uz[j, sl] = vz
        nux[j, sl] = -vx
        nuy[j, sl] = -vy
        nuz[j, sl] = -vz
      _scatter_row(j)

      @pl.when(j >= LAG)
      def _():
        _drain_row(j - LAG)

    for r in range(LAG):
      _drain_row(CROWS - LAG + r)

    pltpu.sync_copy(
        (ux, uy, uz),
        (u_hbm.at[0].at[pl.ds(rbase, CROWS)],
         u_hbm.at[1].at[pl.ds(rbase, CROWS)],
         u_hbm.at[2].at[pl.ds(rbase, CROWS)]),
    )

  plsc.subcore_barrier()
  nsl = pl.ds(sid * NSL, NSL)
  pltpu.sync_copy(accx.at[nsl], du_hbm.at[cid].at[0].at[nsl])
  pltpu.sync_copy(accy.at[nsl], du_hbm.at[cid].at[1].at[nsl])
  pltpu.sync_copy(accz.at[nsl], du_hbm.at[cid].at[2].at[nsl])


@functools.partial(
    pl.kernel,
    out_type=(
        jax.ShapeDtypeStruct((ROWS, B), jnp.float32),
        jax.ShapeDtypeStruct((NPAD,), jnp.float32),
        jax.ShapeDtypeStruct((3, NPAD), jnp.float32),
    ),
    mesh=_mesh,
    compiler_params=_sc_params,
    scratch_types=dict(
        dx=pltpu.VMEM((NPAD,), jnp.float32),
        dy=pltpu.VMEM((NPAD,), jnp.float32),
        dz=pltpu.VMEM((NPAD,), jnp.float32),
        tx=pltpu.VMEM((NPAD,), jnp.float32),
        ty=pltpu.VMEM((NPAD,), jnp.float32),
        tz=pltpu.VMEM((NPAD,), jnp.float32),
        ridx=pltpu.VMEM((CROWS, B), jnp.int32),
        cidx=pltpu.VMEM((CROWS, B), jnp.int32),
        uxv=pltpu.VMEM((CROWS, B), jnp.float32),
        uyv=pltpu.VMEM((CROWS, B), jnp.float32),
        uzv=pltpu.VMEM((CROWS, B), jnp.float32),
        dih=pltpu.VMEM((CROWS, B), jnp.float32),
        angv=pltpu.VMEM((NSL,), jnp.float32),
    ),
)
def _sc_pass_b(row_hbm, col_hbm, u_hbm, du_hbm, dih_hbm, ang_hbm, dsum_hbm, *,
               dx, dy, dz, tx, ty, tz, ridx, cidx, uxv, uyv, uzv, dih, angv):
  cid = lax.axis_index("c")
  sid = lax.axis_index("s")
  wid = sid * NC + cid

  pltpu.sync_copy(
      (du_hbm.at[0].at[0], du_hbm.at[0].at[1], du_hbm.at[0].at[2],
       du_hbm.at[1].at[0], du_hbm.at[1].at[1], du_hbm.at[1].at[2]),
      (dx, dy, dz, tx, ty, tz),
  )

  @pl.loop(0, NPAD // L)
  def _add_loop(i):
    sl = pl.ds(i * L, L)
    dx[sl] = dx[sl] + tx[sl]
    dy[sl] = dy[sl] + ty[sl]
    dz[sl] = dz[sl] + tz[sl]

  @pl.when(cid == 0)
  def _emit_nodes():
    base = sid * NSL

    @pl.loop(0, NSL // L)
    def _ang_loop(i):
      sl = pl.ds(i * L, L)
      vdx = dx[pl.ds(base + i * L, L)]
      vdy = dy[pl.ds(base + i * L, L)]
      vdz = dz[pl.ds(base + i * L, L)]
      angv[sl] = vdx * vdx + vdy * vdy + vdz * vdz

    nsl = pl.ds(base, NSL)
    pltpu.sync_copy(angv, ang_hbm.at[nsl])
    pltpu.sync_copy(dx.at[nsl], dsum_hbm.at[0].at[nsl])
    pltpu.sync_copy(dy.at[nsl], dsum_hbm.at[1].at[nsl])
    pltpu.sync_copy(dz.at[nsl], dsum_hbm.at[2].at[nsl])

  for ch in range(NCHUNK):
    rbase = wid * RPT + ch * CROWS
    rsl = pl.ds(rbase, CROWS)
    pltpu.sync_copy(
        (row_hbm.at[rsl], col_hbm.at[rsl], u_hbm.at[0].at[rsl],
         u_hbm.at[1].at[rsl], u_hbm.at[2].at[rsl]),
        (ridx, cidx, uxv, uyv, uzv),
    )

    @pl.loop(0, CROWS)
    def _row_loop(j):
      for o in range(B // L):
        sl = pl.ds(o * L, L)
        r16 = ridx[j, sl]
        c16 = cidx[j, sl]
        kx = uxv[j, sl]
        ky = uyv[j, sl]
        kz = uzv[j, sl]
        vix = plsc.load_gather(dx, [r16])
        viy = plsc.load_gather(dy, [r16])
        viz = plsc.load_gather(dz, [r16])
        vjx = plsc.load_gather(dx, [c16])
        vjy = plsc.load_gather(dy, [c16])
        vjz = plsc.load_gather(dz, [c16])
        a = vix * kx + viy * ky + viz * kz
        b = vjx * kx + vjy * ky + vjz * kz
        ss = kx * kx + ky * ky + kz * kz
        dot = vix * vjx + viy * vjy + viz * vjz
        dih[j, sl] = dot - a * b * (jnp.float32(2.0) - ss)

    pltpu.sync_copy(dih, dih_hbm.at[pl.ds(rbase, CROWS)])


def _tc_broadcast(xflat, nrows, bcr, w):
  g = nrows // (bcr * w)

  def body(x_ref, o_ref):
    x = x_ref[0]
    o_ref[...] = jnp.broadcast_to(
        x[:, :, None], (bcr, w, HIDDEN)
    ).reshape(bcr * w, HIDDEN)

  return pl.pallas_call(
      body,
      grid=(g,),
      in_specs=[pl.BlockSpec((1, bcr, w), lambda i: (i, 0, 0))],
      out_specs=pl.BlockSpec((bcr * w, HIDDEN), lambda i: (i, 0)),
      out_shape=jax.ShapeDtypeStruct((nrows, HIDDEN), jnp.float32),
  )(xflat.reshape(g, bcr, w))


@jax.jit
def kernel(pos, edge_index, vector_features):
  del vector_features
  pos_t = jnp.zeros((3, NPAD), jnp.float32).at[:, :N_NODES].set(pos.T)
  row2d = edge_index[0].reshape(ROWS, B)
  col2d = edge_index[1].reshape(ROWS, B)

  u_hbm, du_part = _sc_pass_a(pos_t, row2d, col2d)
  dih, ang, dsum = _sc_pass_b(row2d, col2d, u_hbm, du_part)

  dihedral_info = _tc_broadcast(dih.reshape(N_EDGES), N_EDGES, 25, 128)
  angular_info = _tc_broadcast(ang[:N_NODES], N_NODES, 5, 80)

  direction_units = dsum[:, :N_NODES].T
  return (angular_info, dihedral_info, direction_units)


# speedup vs baseline: 23.9235x; 1.0595x over previous
import functools

import jax
import jax.numpy as jnp
from jax import lax
from jax.experimental import pallas as pl
from jax.experimental.pallas import tpu as pltpu
from jax.experimental.pallas import tpu_sc as plsc

N_NODES = 10000
N_EDGES = 320000
HIDDEN = 128

NPAD = 10240
L = 16
NC = 2
NS = 16
NW = NC * NS
B = 80
ROWS = N_EDGES // B
RPT = ROWS // NW
CROWS = 25
NCHUNK = RPT // CROWS
NSL = NPAD // NS

_mesh = plsc.VectorSubcoreMesh(core_axis_name="c", subcore_axis_name="s")
_sc_params = pltpu.CompilerParams(use_tc_tiling_on_sc=False, needs_layout_passes=False)


def _rsqrt(s):
  i = plsc.bitcast(s, jnp.int32)
  i = jnp.int32(0x5F3759DF) - lax.shift_right_logical(i, 1)
  y = plsc.bitcast(i, jnp.float32)
  hs = s * jnp.float32(0.5)
  for _ in range(3):
    y = y * (jnp.float32(1.5) - hs * y * y)
  return y


@functools.partial(
    pl.kernel,
    out_type=(
        jax.ShapeDtypeStruct((3, ROWS, B), jnp.float32),
        jax.ShapeDtypeStruct((NC, 3, NPAD), jnp.float32),
    ),
    mesh=_mesh,
    compiler_params=_sc_params,
    scratch_types=dict(
        px=pltpu.VMEM((NPAD,), jnp.float32),
        py=pltpu.VMEM((NPAD,), jnp.float32),
        pz=pltpu.VMEM((NPAD,), jnp.float32),
        ridx=pltpu.VMEM((CROWS, B), jnp.int32),
        cidx=pltpu.VMEM((CROWS, B), jnp.int32),
        ux=pltpu.VMEM((CROWS, B), jnp.float32),
        uy=pltpu.VMEM((CROWS, B), jnp.float32),
        uz=pltpu.VMEM((CROWS, B), jnp.float32),
        nux=pltpu.VMEM((CROWS, B), jnp.float32),
        nuy=pltpu.VMEM((CROWS, B), jnp.float32),
        nuz=pltpu.VMEM((CROWS, B), jnp.float32),
        zb=pltpu.VMEM((NSL,), jnp.float32),
        accx=pltpu.VMEM_SHARED((NPAD,), jnp.float32),
        accy=pltpu.VMEM_SHARED((NPAD,), jnp.float32),
        accz=pltpu.VMEM_SHARED((NPAD,), jnp.float32),
        sem=pltpu.SemaphoreType.DMA,
    ),
)
def _sc_pass_a(pos_hbm, row_hbm, col_hbm, u_hbm, du_hbm, *, px, py, pz,
               ridx, cidx, ux, uy, uz, nux, nuy, nuz, zb, accx, accy, accz,
               sem):
  cid = lax.axis_index("c")
  sid = lax.axis_index("s")
  wid = sid * NC + cid

  pltpu.sync_copy(pos_hbm.at[0], px)
  pltpu.sync_copy(pos_hbm.at[1], py)
  pltpu.sync_copy(pos_hbm.at[2], pz)

  for i in range(NSL // L):
    zb[pl.ds(i * L, L)] = jnp.zeros((L,), jnp.float32)
  pltpu.sync_copy(zb, accx.at[pl.ds(sid * NSL, NSL)])
  pltpu.sync_copy(zb, accy.at[pl.ds(sid * NSL, NSL)])
  pltpu.sync_copy(zb, accz.at[pl.ds(sid * NSL, NSL)])
  plsc.subcore_barrier()

  def _scatter_row(j):
    pltpu.async_copy(ux.at[j], accx.at[ridx.at[j]], sem, add=True)
    pltpu.async_copy(uy.at[j], accy.at[ridx.at[j]], sem, add=True)
    pltpu.async_copy(uz.at[j], accz.at[ridx.at[j]], sem, add=True)
    pltpu.async_copy(nux.at[j], accx.at[cidx.at[j]], sem, add=True)
    pltpu.async_copy(nuy.at[j], accy.at[cidx.at[j]], sem, add=True)
    pltpu.async_copy(nuz.at[j], accz.at[cidx.at[j]], sem, add=True)

  def _drain_row(j):
    pltpu.make_async_copy(ux.at[j], accx.at[ridx.at[j]], sem).wait()
    pltpu.make_async_copy(uy.at[j], accy.at[ridx.at[j]], sem).wait()
    pltpu.make_async_copy(uz.at[j], accz.at[ridx.at[j]], sem).wait()
    pltpu.make_async_copy(nux.at[j], accx.at[cidx.at[j]], sem).wait()
    pltpu.make_async_copy(nuy.at[j], accy.at[cidx.at[j]], sem).wait()
    pltpu.make_async_copy(nuz.at[j], accz.at[cidx.at[j]], sem).wait()

  LAG = 5

  for ch in range(NCHUNK):
    rbase = wid * RPT + ch * CROWS
    pltpu.sync_copy(
        (row_hbm.at[pl.ds(rbase, CROWS)], col_hbm.at[pl.ds(rbase, CROWS)]),
        (ridx, cidx),
    )

    @pl.loop(0, CROWS)
    def _row_loop(j):
      for o in range(B // L):
        sl = pl.ds(o * L, L)
        r16 = ridx[j, sl]
        c16 = cidx[j, sl]
        ex = plsc.load_gather(px, [c16]) - plsc.load_gather(px, [r16])
        ey = plsc.load_gather(py, [c16]) - plsc.load_gather(py, [r16])
        ez = plsc.load_gather(pz, [c16]) - plsc.load_gather(pz, [r16])
        s = ex * ex + ey * ey + ez * ez
        norm = s * _rsqrt(s)
        inv = jnp.float32(1.0) / (norm + jnp.float32(1e-8))
        vx = ex * inv
        vy = ey * inv
        vz = ez * inv
        ux[j, sl] = vx
        uy[j, sl] = vy
        uz[j, sl] = vz
        nux[j, sl] = -vx
        nuy[j, sl] = -vy
        nuz[j, sl] = -vz
      _scatter_row(j)

      @pl.when(j >= LAG)
      def _():
        _drain_row(j - LAG)

    for r in range(LAG):
      _drain_row(CROWS - LAG + r)

    pltpu.sync_copy(
        (ux, uy, uz),
        (u_hbm.at[0].at[pl.ds(rbase, CROWS)],
         u_hbm.at[1].at[pl.ds(rbase, CROWS)],
         u_hbm.at[2].at[pl.ds(rbase, CROWS)]),
    )

  plsc.subcore_barrier()
  nsl = pl.ds(sid * NSL, NSL)
  pltpu.sync_copy(accx.at[nsl], du_hbm.at[cid].at[0].at[nsl])
  pltpu.sync_copy(accy.at[nsl], du_hbm.at[cid].at[1].at[nsl])
  pltpu.sync_copy(accz.at[nsl], du_hbm.at[cid].at[2].at[nsl])


def _make_sc_pass_b(row_lo, rpt, nchunk, emit_nodes):
  assert rpt == nchunk * CROWS
  out_type = [jax.ShapeDtypeStruct((NW * rpt, B), jnp.float32)]
  if emit_nodes:
    out_type += [
        jax.ShapeDtypeStruct((NPAD,), jnp.float32),
        jax.ShapeDtypeStruct((3, NPAD), jnp.float32),
    ]

  @functools.partial(
      pl.kernel,
      out_type=tuple(out_type),
      mesh=_mesh,
      compiler_params=_sc_params,
      scratch_types=dict(
          dx=pltpu.VMEM((NPAD,), jnp.float32),
          dy=pltpu.VMEM((NPAD,), jnp.float32),
          dz=pltpu.VMEM((NPAD,), jnp.float32),
          tx=pltpu.VMEM((NPAD,), jnp.float32),
          ty=pltpu.VMEM((NPAD,), jnp.float32),
          tz=pltpu.VMEM((NPAD,), jnp.float32),
          ridx=pltpu.VMEM((CROWS, B), jnp.int32),
          cidx=pltpu.VMEM((CROWS, B), jnp.int32),
          uxv=pltpu.VMEM((CROWS, B), jnp.float32),
          uyv=pltpu.VMEM((CROWS, B), jnp.float32),
          uzv=pltpu.VMEM((CROWS, B), jnp.float32),
          dih=pltpu.VMEM((CROWS, B), jnp.float32),
          angv=pltpu.VMEM((NSL,), jnp.float32),
      ),
  )
  def _pass_b(row_hbm, col_hbm, u_hbm, du_hbm, dih_hbm, *out_refs,
              dx, dy, dz, tx, ty, tz, ridx, cidx, uxv, uyv, uzv, dih, angv):
    cid = lax.axis_index("c")
    sid = lax.axis_index("s")
    wid = sid * NC + cid

    pltpu.sync_copy(
        (du_hbm.at[0].at[0], du_hbm.at[0].at[1], du_hbm.at[0].at[2],
         du_hbm.at[1].at[0], du_hbm.at[1].at[1], du_hbm.at[1].at[2]),
        (dx, dy, dz, tx, ty, tz),
    )

    @pl.loop(0, NPAD // L)
    def _add_loop(i):
      sl = pl.ds(i * L, L)
      dx[sl] = dx[sl] + tx[sl]
      dy[sl] = dy[sl] + ty[sl]
      dz[sl] = dz[sl] + tz[sl]

    if emit_nodes:
      ang_hbm, dsum_hbm = out_refs

      @pl.when(cid == 0)
      def _emit_nodes():
        base = sid * NSL

        @pl.loop(0, NSL // L)
        def _ang_loop(i):
          sl = pl.ds(i * L, L)
          vdx = dx[pl.ds(base + i * L, L)]
          vdy = dy[pl.ds(base + i * L, L)]
          vdz = dz[pl.ds(base + i * L, L)]
          angv[sl] = vdx * vdx + vdy * vdy + vdz * vdz

        nsl = pl.ds(base, NSL)
        pltpu.sync_copy(
            (angv, dx.at[nsl], dy.at[nsl], dz.at[nsl]),
            (ang_hbm.at[nsl], dsum_hbm.at[0].at[nsl],
             dsum_hbm.at[1].at[nsl], dsum_hbm.at[2].at[nsl]),
        )

    for ch in range(nchunk):
      lbase = wid * rpt + ch * CROWS
      rsl = pl.ds(row_lo + lbase, CROWS)
      pltpu.sync_copy(
          (row_hbm.at[rsl], col_hbm.at[rsl], u_hbm.at[0].at[rsl],
           u_hbm.at[1].at[rsl], u_hbm.at[2].at[rsl]),
          (ridx, cidx, uxv, uyv, uzv),
      )

      @pl.loop(0, CROWS)
      def _row_loop(j):
        for o in range(B // L):
          sl = pl.ds(o * L, L)
          r16 = ridx[j, sl]
          c16 = cidx[j, sl]
          kx = uxv[j, sl]
          ky = uyv[j, sl]
          kz = uzv[j, sl]
          vix = plsc.load_gather(dx, [r16])
          viy = plsc.load_gather(dy, [r16])
          viz = plsc.load_gather(dz, [r16])
          vjx = plsc.load_gather(dx, [c16])
          vjy = plsc.load_gather(dy, [c16])
          vjz = plsc.load_gather(dz, [c16])
          a = vix * kx + viy * ky + viz * kz
          b = vjx * kx + vjy * ky + vjz * kz
          ss = kx * kx + ky * ky + kz * kz
          dot = vix * vjx + viy * vjy + viz * vjz
          dih[j, sl] = dot - a * b * (jnp.float32(2.0) - ss)

      pltpu.sync_copy(dih, dih_hbm.at[pl.ds(lbase, CROWS)])

  return _pass_b


ROWS1 = NW * 2 * CROWS
ROWS2 = ROWS - ROWS1
_sc_pass_b1 = _make_sc_pass_b(0, 2 * CROWS, 2, True)
_sc_pass_b2 = _make_sc_pass_b(ROWS1, 3 * CROWS, 3, False)


def _tc_broadcast(xflat, nrows, bcr, w):
  g = nrows // (bcr * w)

  def body(x_ref, o_ref):
    x = x_ref[0]
    o_ref[...] = jnp.broadcast_to(
        x[:, :, None], (bcr, w, HIDDEN)
    ).reshape(bcr * w, HIDDEN)

  return pl.pallas_call(
      body,
      grid=(g,),
      in_specs=[pl.BlockSpec((1, bcr, w), lambda i: (i, 0, 0))],
      out_specs=pl.BlockSpec((bcr * w, HIDDEN), lambda i: (i, 0)),
      out_shape=jax.ShapeDtypeStruct((nrows, HIDDEN), jnp.float32),
  )(xflat.reshape(g, bcr, w))


_BCR = 25
_BW = _BCR * 128


def _tc_broadcast_part1(xflat):
  g = (ROWS1 * B) // _BW

  def body(x_ref, o_ref):
    x = x_ref[0]
    o_ref[...] = jnp.broadcast_to(
        x[:, :, None], (_BCR, 128, HIDDEN)
    ).reshape(_BW, HIDDEN)

  return pl.pallas_call(
      body,
      grid=(g,),
      in_specs=[pl.BlockSpec((1, _BCR, 128), lambda i: (i, 0, 0))],
      out_specs=pl.BlockSpec((_BW, HIDDEN), lambda i: (i, 0)),
      out_shape=jax.ShapeDtypeStruct((N_EDGES, HIDDEN), jnp.float32),
  )(xflat.reshape(g, _BCR, 128))


def _tc_broadcast_part2(xflat, part1):
  g = (ROWS2 * B) // _BW
  off = (ROWS1 * B) // _BW

  def body(x_ref, a_ref, o_ref):
    del a_ref
    x = x_ref[0]
    o_ref[...] = jnp.broadcast_to(
        x[:, :, None], (_BCR, 128, HIDDEN)
    ).reshape(_BW, HIDDEN)

  return pl.pallas_call(
      body,
      grid=(g,),
      in_specs=[
          pl.BlockSpec((1, _BCR, 128), lambda i: (i, 0, 0)),
          pl.BlockSpec(memory_space=pl.ANY),
      ],
      out_specs=pl.BlockSpec((_BW, HIDDEN), lambda i: (i + off, 0)),
      out_shape=jax.ShapeDtypeStruct((N_EDGES, HIDDEN), jnp.float32),
      input_output_aliases={1: 0},
  )(xflat.reshape(g, _BCR, 128), part1)


@jax.jit
def kernel(pos, edge_index, vector_features):
  del vector_features
  pos_t = jnp.zeros((3, NPAD), jnp.float32).at[:, :N_NODES].set(pos.T)
  row2d = edge_index[0].reshape(ROWS, B)
  col2d = edge_index[1].reshape(ROWS, B)

  u_hbm, du_part = _sc_pass_a(pos_t, row2d, col2d)
  dih1, ang, dsum = _sc_pass_b1(row2d, col2d, u_hbm, du_part)
  dih2 = _sc_pass_b2(row2d, col2d, u_hbm, du_part)
  if isinstance(dih2, (tuple, list)):
    dih2 = dih2[0]

  part1 = _tc_broadcast_part1(dih1.reshape(ROWS1 * B))
  dihedral_info = _tc_broadcast_part2(dih2.reshape(ROWS2 * B), part1)
  angular_info = _tc_broadcast(ang[:N_NODES], N_NODES, 25, 80)

  direction_units = dsum[:, :N_NODES].T
  return (angular_info, dihedral_info, direction_units)


# speedup vs baseline: 25.3596x; 1.0600x over previous
import functools

import jax
import jax.numpy as jnp
from jax import lax
from jax.experimental import pallas as pl
from jax.experimental.pallas import tpu as pltpu
from jax.experimental.pallas import tpu_sc as plsc

N_NODES = 10000
N_EDGES = 320000
HIDDEN = 128

NPAD = 10240
L = 16
NC = 2
NS = 16
NW = NC * NS
B = 80
ROWS = N_EDGES // B
RPT = ROWS // NW
CROWS = 25
NCHUNK = RPT // CROWS
NSL = NPAD // NS

_mesh = plsc.VectorSubcoreMesh(core_axis_name="c", subcore_axis_name="s")
_sc_params = pltpu.CompilerParams(use_tc_tiling_on_sc=False, needs_layout_passes=False)


def _rsqrt(s):
  i = plsc.bitcast(s, jnp.int32)
  i = jnp.int32(0x5F3759DF) - lax.shift_right_logical(i, 1)
  y = plsc.bitcast(i, jnp.float32)
  hs = s * jnp.float32(0.5)
  for _ in range(3):
    y = y * (jnp.float32(1.5) - hs * y * y)
  return y


@functools.partial(
    pl.kernel,
    out_type=(
        jax.ShapeDtypeStruct((3, ROWS, B), jnp.float32),
        jax.ShapeDtypeStruct((NC, 3, NPAD), jnp.float32),
    ),
    mesh=_mesh,
    compiler_params=_sc_params,
    scratch_types=dict(
        px=pltpu.VMEM((NPAD,), jnp.float32),
        py=pltpu.VMEM((NPAD,), jnp.float32),
        pz=pltpu.VMEM((NPAD,), jnp.float32),
        ridx=pltpu.VMEM((CROWS, B), jnp.int32),
        cidx=pltpu.VMEM((CROWS, B), jnp.int32),
        ux=pltpu.VMEM((CROWS, B), jnp.float32),
        uy=pltpu.VMEM((CROWS, B), jnp.float32),
        uz=pltpu.VMEM((CROWS, B), jnp.float32),
        nux=pltpu.VMEM((CROWS, B), jnp.float32),
        nuy=pltpu.VMEM((CROWS, B), jnp.float32),
        nuz=pltpu.VMEM((CROWS, B), jnp.float32),
        zb=pltpu.VMEM((NSL,), jnp.float32),
        accx=pltpu.VMEM_SHARED((NPAD,), jnp.float32),
        accy=pltpu.VMEM_SHARED((NPAD,), jnp.float32),
        accz=pltpu.VMEM_SHARED((NPAD,), jnp.float32),
        sem=pltpu.SemaphoreType.DMA,
    ),
)
def _sc_pass_a(pos_hbm, row_hbm, col_hbm, u_hbm, du_hbm, *, px, py, pz,
               ridx, cidx, ux, uy, uz, nux, nuy, nuz, zb, accx, accy, accz,
               sem):
  cid = lax.axis_index("c")
  sid = lax.axis_index("s")
  wid = sid * NC + cid

  pltpu.sync_copy(pos_hbm.at[0], px)
  pltpu.sync_copy(pos_hbm.at[1], py)
  pltpu.sync_copy(pos_hbm.at[2], pz)

  for i in range(NSL // L):
    zb[pl.ds(i * L, L)] = jnp.zeros((L,), jnp.float32)
  pltpu.sync_copy(zb, accx.at[pl.ds(sid * NSL, NSL)])
  pltpu.sync_copy(zb, accy.at[pl.ds(sid * NSL, NSL)])
  pltpu.sync_copy(zb, accz.at[pl.ds(sid * NSL, NSL)])
  plsc.subcore_barrier()

  def _scatter_row(j):
    pltpu.async_copy(ux.at[j], accx.at[ridx.at[j]], sem, add=True)
    pltpu.async_copy(uy.at[j], accy.at[ridx.at[j]], sem, add=True)
    pltpu.async_copy(uz.at[j], accz.at[ridx.at[j]], sem, add=True)
    pltpu.async_copy(nux.at[j], accx.at[cidx.at[j]], sem, add=True)
    pltpu.async_copy(nuy.at[j], accy.at[cidx.at[j]], sem, add=True)
    pltpu.async_copy(nuz.at[j], accz.at[cidx.at[j]], sem, add=True)

  def _drain_row(j):
    pltpu.make_async_copy(ux.at[j], accx.at[ridx.at[j]], sem).wait()
    pltpu.make_async_copy(uy.at[j], accy.at[ridx.at[j]], sem).wait()
    pltpu.make_async_copy(uz.at[j], accz.at[ridx.at[j]], sem).wait()
    pltpu.make_async_copy(nux.at[j], accx.at[cidx.at[j]], sem).wait()
    pltpu.make_async_copy(nuy.at[j], accy.at[cidx.at[j]], sem).wait()
    pltpu.make_async_copy(nuz.at[j], accz.at[cidx.at[j]], sem).wait()

  LAG = 5

  for ch in range(NCHUNK):
    rbase = wid * RPT + ch * CROWS
    pltpu.sync_copy(
        (row_hbm.at[pl.ds(rbase, CROWS)], col_hbm.at[pl.ds(rbase, CROWS)]),
        (ridx, cidx),
    )

    @pl.loop(0, CROWS)
    def _row_loop(j):
      for o in range(B // L):
        sl = pl.ds(o * L, L)
        r16 = ridx[j, sl]
        c16 = cidx[j, sl]
        ex = plsc.load_gather(px, [c16]) - plsc.load_gather(px, [r16])
        ey = plsc.load_gather(py, [c16]) - plsc.load_gather(py, [r16])
        ez = plsc.load_gather(pz, [c16]) - plsc.load_gather(pz, [r16])
        s = ex * ex + ey * ey + ez * ez
        norm = s * _rsqrt(s)
        inv = jnp.float32(1.0) / (norm + jnp.float32(1e-8))
        vx = ex * inv
        vy = ey * inv
        vz = ez * inv
        ux[j, sl] = vx
        uy[j, sl] = vy
        uz[j, sl] = vz
        nux[j, sl] = -vx
        nuy[j, sl] = -vy
        nuz[j, sl] = -vz
      _scatter_row(j)

      @pl.when(j >= LAG)
      def _():
        _drain_row(j - LAG)

    for r in range(LAG):
      _drain_row(CROWS - LAG + r)

    pltpu.sync_copy(
        (ux, uy, uz),
        (u_hbm.at[0].at[pl.ds(rbase, CROWS)],
         u_hbm.at[1].at[pl.ds(rbase, CROWS)],
         u_hbm.at[2].at[pl.ds(rbase, CROWS)]),
    )

  plsc.subcore_barrier()
  nsl = pl.ds(sid * NSL, NSL)
  pltpu.sync_copy(accx.at[nsl], du_hbm.at[cid].at[0].at[nsl])
  pltpu.sync_copy(accy.at[nsl], du_hbm.at[cid].at[1].at[nsl])
  pltpu.sync_copy(accz.at[nsl], du_hbm.at[cid].at[2].at[nsl])


def _make_sc_pass_b(row_lo, rpt, nchunk, emit_nodes):
  assert rpt == nchunk * CROWS
  out_type = [jax.ShapeDtypeStruct((NW * rpt, B), jnp.float32)]
  if emit_nodes:
    out_type += [
        jax.ShapeDtypeStruct((NPAD,), jnp.float32),
        jax.ShapeDtypeStruct((3, NPAD), jnp.float32),
    ]

  @functools.partial(
      pl.kernel,
      out_type=tuple(out_type),
      mesh=_mesh,
      compiler_params=_sc_params,
      scratch_types=dict(
          dx=pltpu.VMEM((NPAD,), jnp.float32),
          dy=pltpu.VMEM((NPAD,), jnp.float32),
          dz=pltpu.VMEM((NPAD,), jnp.float32),
          tx=pltpu.VMEM((NPAD,), jnp.float32),
          ty=pltpu.VMEM((NPAD,), jnp.float32),
          tz=pltpu.VMEM((NPAD,), jnp.float32),
          ridx=pltpu.VMEM((CROWS, B), jnp.int32),
          cidx=pltpu.VMEM((CROWS, B), jnp.int32),
          uxv=pltpu.VMEM((CROWS, B), jnp.float32),
          uyv=pltpu.VMEM((CROWS, B), jnp.float32),
          uzv=pltpu.VMEM((CROWS, B), jnp.float32),
          dih=pltpu.VMEM((CROWS, B), jnp.float32),
          angv=pltpu.VMEM((NSL,), jnp.float32),
      ),
  )
  def _pass_b(row_hbm, col_hbm, u_hbm, du_hbm, dih_hbm, *out_refs,
              dx, dy, dz, tx, ty, tz, ridx, cidx, uxv, uyv, uzv, dih, angv):
    cid = lax.axis_index("c")
    sid = lax.axis_index("s")
    wid = sid * NC + cid

    pltpu.sync_copy(
        (du_hbm.at[0].at[0], du_hbm.at[0].at[1], du_hbm.at[0].at[2],
         du_hbm.at[1].at[0], du_hbm.at[1].at[1], du_hbm.at[1].at[2]),
        (dx, dy, dz, tx, ty, tz),
    )

    @pl.loop(0, NPAD // L)
    def _add_loop(i):
      sl = pl.ds(i * L, L)
      dx[sl] = dx[sl] + tx[sl]
      dy[sl] = dy[sl] + ty[sl]
      dz[sl] = dz[sl] + tz[sl]

    if emit_nodes:
      ang_hbm, dsum_hbm = out_refs

      @pl.when(cid == 0)
      def _emit_nodes():
        base = sid * NSL

        @pl.loop(0, NSL // L)
        def _ang_loop(i):
          sl = pl.ds(i * L, L)
          vdx = dx[pl.ds(base + i * L, L)]
          vdy = dy[pl.ds(base + i * L, L)]
          vdz = dz[pl.ds(base + i * L, L)]
          angv[sl] = vdx * vdx + vdy * vdy + vdz * vdz

        nsl = pl.ds(base, NSL)
        pltpu.sync_copy(
            (angv, dx.at[nsl], dy.at[nsl], dz.at[nsl]),
            (ang_hbm.at[nsl], dsum_hbm.at[0].at[nsl],
             dsum_hbm.at[1].at[nsl], dsum_hbm.at[2].at[nsl]),
        )

    for ch in range(nchunk):
      lbase = wid * rpt + ch * CROWS
      rsl = pl.ds(row_lo + lbase, CROWS)
      pltpu.sync_copy(
          (row_hbm.at[rsl], col_hbm.at[rsl], u_hbm.at[0].at[rsl],
           u_hbm.at[1].at[rsl], u_hbm.at[2].at[rsl]),
          (ridx, cidx, uxv, uyv, uzv),
      )

      @pl.loop(0, CROWS)
      def _row_loop(j):
        for o in range(B // L):
          sl = pl.ds(o * L, L)
          r16 = ridx[j, sl]
          c16 = cidx[j, sl]
          kx = uxv[j, sl]
          ky = uyv[j, sl]
          kz = uzv[j, sl]
          vix = plsc.load_gather(dx, [r16])
          viy = plsc.load_gather(dy, [r16])
          viz = plsc.load_gather(dz, [r16])
          vjx = plsc.load_gather(dx, [c16])
          vjy = plsc.load_gather(dy, [c16])
          vjz = plsc.load_gather(dz, [c16])
          a = vix * kx + viy * ky + viz * kz
          b = vjx * kx + vjy * ky + vjz * kz
          ss = kx * kx + ky * ky + kz * kz
          dot = vix * vjx + viy * vjy + viz * vjz
          dih[j, sl] = dot - a * b * (jnp.float32(2.0) - ss)

      pltpu.sync_copy(dih, dih_hbm.at[pl.ds(lbase, CROWS)])

  return _pass_b


ROWS1 = NW * CROWS
ROWS2 = ROWS - ROWS1
_sc_pass_b1 = _make_sc_pass_b(0, CROWS, 1, True)
_sc_pass_b2 = _make_sc_pass_b(ROWS1, 4 * CROWS, 4, False)


def _tc_broadcast(xflat, nrows, bcr, w):
  g = nrows // (bcr * w)

  def body(x_ref, o_ref):
    x = x_ref[0]
    o_ref[...] = jnp.broadcast_to(
        x[:, :, None], (bcr, w, HIDDEN)
    ).reshape(bcr * w, HIDDEN)

  return pl.pallas_call(
      body,
      grid=(g,),
      in_specs=[pl.BlockSpec((1, bcr, w), lambda i: (i, 0, 0))],
      out_specs=pl.BlockSpec((bcr * w, HIDDEN), lambda i: (i, 0)),
      out_shape=jax.ShapeDtypeStruct((nrows, HIDDEN), jnp.float32),
  )(xflat.reshape(g, bcr, w))


_BCR = 50
_BW = _BCR * 128


def _tc_broadcast_part1(xflat):
  g = (ROWS1 * B) // _BW

  def body(x_ref, o_ref):
    x = x_ref[0]
    o_ref[...] = jnp.broadcast_to(
        x[:, :, None], (_BCR, 128, HIDDEN)
    ).reshape(_BW, HIDDEN)

  return pl.pallas_call(
      body,
      grid=(g,),
      in_specs=[pl.BlockSpec((1, _BCR, 128), lambda i: (i, 0, 0))],
      out_specs=pl.BlockSpec((_BW, HIDDEN), lambda i: (i, 0)),
      out_shape=jax.ShapeDtypeStruct((N_EDGES, HIDDEN), jnp.float32),
  )(xflat.reshape(g, _BCR, 128))


def _tc_broadcast_part2(xflat, part1):
  g = (ROWS2 * B) // _BW
  off = (ROWS1 * B) // _BW

  def body(x_ref, a_ref, o_ref):
    del a_ref
    x = x_ref[0]
    o_ref[...] = jnp.broadcast_to(
        x[:, :, None], (_BCR, 128, HIDDEN)
    ).reshape(_BW, HIDDEN)

  return pl.pallas_call(
      body,
      grid=(g,),
      in_specs=[
          pl.BlockSpec((1, _BCR, 128), lambda i: (i, 0, 0)),
          pl.BlockSpec(memory_space=pl.ANY),
      ],
      out_specs=pl.BlockSpec((_BW, HIDDEN), lambda i: (i + off, 0)),
      out_shape=jax.ShapeDtypeStruct((N_EDGES, HIDDEN), jnp.float32),
      input_output_aliases={1: 0},
  )(xflat.reshape(g, _BCR, 128), part1)


@jax.jit
def kernel(pos, edge_index, vector_features):
  del vector_features
  pos_t = jnp.zeros((3, NPAD), jnp.float32).at[:, :N_NODES].set(pos.T)
  row2d = edge_index[0].reshape(ROWS, B)
  col2d = edge_index[1].reshape(ROWS, B)

  u_hbm, du_part = _sc_pass_a(pos_t, row2d, col2d)
  dih1, ang, dsum = _sc_pass_b1(row2d, col2d, u_hbm, du_part)
  dih2 = _sc_pass_b2(row2d, col2d, u_hbm, du_part)
  if isinstance(dih2, (tuple, list)):
    dih2 = dih2[0]

  part1 = _tc_broadcast_part1(dih1.reshape(ROWS1 * B))
  dihedral_info = _tc_broadcast_part2(dih2.reshape(ROWS2 * B), part1)
  angular_info = _tc_broadcast(ang[:N_NODES], N_NODES, 25, 80)

  direction_units = dsum[:, :N_NODES].T
  return (angular_info, dihedral_info, direction_units)


# speedup vs baseline: 25.7709x; 1.0162x over previous
import functools

import jax
import jax.numpy as jnp
from jax import lax
from jax.experimental import pallas as pl
from jax.experimental.pallas import tpu as pltpu
from jax.experimental.pallas import tpu_sc as plsc

N_NODES = 10000
N_EDGES = 320000
HIDDEN = 128

NPAD = 10240
L = 16
NC = 2
NS = 16
NW = NC * NS
B = 80
ROWS = N_EDGES // B
RPT = ROWS // NW
CROWS = 25
NCHUNK = RPT // CROWS
NSL = NPAD // NS

_mesh = plsc.VectorSubcoreMesh(core_axis_name="c", subcore_axis_name="s")
_sc_params = pltpu.CompilerParams(use_tc_tiling_on_sc=False, needs_layout_passes=False)


def _rsqrt(s):
  i = plsc.bitcast(s, jnp.int32)
  i = jnp.int32(0x5F3759DF) - lax.shift_right_logical(i, 1)
  y = plsc.bitcast(i, jnp.float32)
  hs = s * jnp.float32(0.5)
  for _ in range(3):
    y = y * (jnp.float32(1.5) - hs * y * y)
  return y


@functools.partial(
    pl.kernel,
    out_type=(
        jax.ShapeDtypeStruct((3, ROWS, B), jnp.float32),
        jax.ShapeDtypeStruct((NC, 3, NPAD), jnp.float32),
    ),
    mesh=_mesh,
    compiler_params=_sc_params,
    scratch_types=dict(
        px=pltpu.VMEM((NPAD,), jnp.float32),
        py=pltpu.VMEM((NPAD,), jnp.float32),
        pz=pltpu.VMEM((NPAD,), jnp.float32),
        ridx=pltpu.VMEM((CROWS, B), jnp.int32),
        cidx=pltpu.VMEM((CROWS, B), jnp.int32),
        ux=pltpu.VMEM((CROWS, B), jnp.float32),
        uy=pltpu.VMEM((CROWS, B), jnp.float32),
        uz=pltpu.VMEM((CROWS, B), jnp.float32),
        nux=pltpu.VMEM((CROWS, B), jnp.float32),
        nuy=pltpu.VMEM((CROWS, B), jnp.float32),
        nuz=pltpu.VMEM((CROWS, B), jnp.float32),
        zb=pltpu.VMEM((NSL,), jnp.float32),
        accx=pltpu.VMEM_SHARED((NPAD,), jnp.float32),
        accy=pltpu.VMEM_SHARED((NPAD,), jnp.float32),
        accz=pltpu.VMEM_SHARED((NPAD,), jnp.float32),
        sem=pltpu.SemaphoreType.DMA,
    ),
)
def _sc_pass_a(pos_hbm, row_hbm, col_hbm, u_hbm, du_hbm, *, px, py, pz,
               ridx, cidx, ux, uy, uz, nux, nuy, nuz, zb, accx, accy, accz,
               sem):
  cid = lax.axis_index("c")
  sid = lax.axis_index("s")
  wid = sid * NC + cid

  pltpu.sync_copy(pos_hbm.at[0], px)
  pltpu.sync_copy(pos_hbm.at[1], py)
  pltpu.sync_copy(pos_hbm.at[2], pz)

  for i in range(NSL // L):
    zb[pl.ds(i * L, L)] = jnp.zeros((L,), jnp.float32)
  pltpu.sync_copy(zb, accx.at[pl.ds(sid * NSL, NSL)])
  pltpu.sync_copy(zb, accy.at[pl.ds(sid * NSL, NSL)])
  pltpu.sync_copy(zb, accz.at[pl.ds(sid * NSL, NSL)])
  plsc.subcore_barrier()

  def _scatter_row(j):
    pltpu.async_copy(ux.at[j], accx.at[ridx.at[j]], sem, add=True)
    pltpu.async_copy(uy.at[j], accy.at[ridx.at[j]], sem, add=True)
    pltpu.async_copy(uz.at[j], accz.at[ridx.at[j]], sem, add=True)
    pltpu.async_copy(nux.at[j], accx.at[cidx.at[j]], sem, add=True)
    pltpu.async_copy(nuy.at[j], accy.at[cidx.at[j]], sem, add=True)
    pltpu.async_copy(nuz.at[j], accz.at[cidx.at[j]], sem, add=True)

  def _drain_row(j):
    pltpu.make_async_copy(ux.at[j], accx.at[ridx.at[j]], sem).wait()
    pltpu.make_async_copy(uy.at[j], accy.at[ridx.at[j]], sem).wait()
    pltpu.make_async_copy(uz.at[j], accz.at[ridx.at[j]], sem).wait()
    pltpu.make_async_copy(nux.at[j], accx.at[cidx.at[j]], sem).wait()
    pltpu.make_async_copy(nuy.at[j], accy.at[cidx.at[j]], sem).wait()
    pltpu.make_async_copy(nuz.at[j], accz.at[cidx.at[j]], sem).wait()

  LAG = 5

  for ch in range(NCHUNK):
    rbase = wid * RPT + ch * CROWS
    pltpu.sync_copy(
        (row_hbm.at[pl.ds(rbase, CROWS)], col_hbm.at[pl.ds(rbase, CROWS)]),
        (ridx, cidx),
    )

    @pl.loop(0, CROWS)
    def _row_loop(j):
      for o in range(B // L):
        sl = pl.ds(o * L, L)
        r16 = ridx[j, sl]
        c16 = cidx[j, sl]
        ex = plsc.load_gather(px, [c16]) - plsc.load_gather(px, [r16])
        ey = plsc.load_gather(py, [c16]) - plsc.load_gather(py, [r16])
        ez = plsc.load_gather(pz, [c16]) - plsc.load_gather(pz, [r16])
        s = ex * ex + ey * ey + ez * ez
        norm = s * _rsqrt(s)
        inv = jnp.float32(1.0) / (norm + jnp.float32(1e-8))
        vx = ex * inv
        vy = ey * inv
        vz = ez * inv
        ux[j, sl] = vx
        uy[j, sl] = vy
        uz[j, sl] = vz
        nux[j, sl] = -vx
        nuy[j, sl] = -vy
        nuz[j, sl] = -vz
      _scatter_row(j)

      @pl.when(j >= LAG)
      def _():
        _drain_row(j - LAG)

    for r in range(LAG):
      _drain_row(CROWS - LAG + r)

    pltpu.sync_copy(
        (ux, uy, uz),
        (u_hbm.at[0].at[pl.ds(rbase, CROWS)],
         u_hbm.at[1].at[pl.ds(rbase, CROWS)],
         u_hbm.at[2].at[pl.ds(rbase, CROWS)]),
    )

  plsc.subcore_barrier()
  nsl = pl.ds(sid * NSL, NSL)
  pltpu.sync_copy(accx.at[nsl], du_hbm.at[cid].at[0].at[nsl])
  pltpu.sync_copy(accy.at[nsl], du_hbm.at[cid].at[1].at[nsl])
  pltpu.sync_copy(accz.at[nsl], du_hbm.at[cid].at[2].at[nsl])


def _make_sc_pass_b(row_lo, rpt, nchunk, emit_nodes, bcrows):
  assert rpt == nchunk * bcrows
  out_type = [jax.ShapeDtypeStruct((NW * rpt, B), jnp.float32)]
  if emit_nodes:
    out_type += [
        jax.ShapeDtypeStruct((NPAD,), jnp.float32),
        jax.ShapeDtypeStruct((3, NPAD), jnp.float32),
    ]

  @functools.partial(
      pl.kernel,
      out_type=tuple(out_type),
      mesh=_mesh,
      compiler_params=_sc_params,
      scratch_types=dict(
          dx=pltpu.VMEM((NPAD,), jnp.float32),
          dy=pltpu.VMEM((NPAD,), jnp.float32),
          dz=pltpu.VMEM((NPAD,), jnp.float32),
          tx=pltpu.VMEM((NPAD,), jnp.float32),
          ty=pltpu.VMEM((NPAD,), jnp.float32),
          tz=pltpu.VMEM((NPAD,), jnp.float32),
          ridx=pltpu.VMEM((bcrows, B), jnp.int32),
          cidx=pltpu.VMEM((bcrows, B), jnp.int32),
          uxv=pltpu.VMEM((bcrows, B), jnp.float32),
          uyv=pltpu.VMEM((bcrows, B), jnp.float32),
          uzv=pltpu.VMEM((bcrows, B), jnp.float32),
          dih=pltpu.VMEM((bcrows, B), jnp.float32),
          angv=pltpu.VMEM((NSL,), jnp.float32),
      ),
  )
  def _pass_b(row_hbm, col_hbm, u_hbm, du_hbm, dih_hbm, *out_refs,
              dx, dy, dz, tx, ty, tz, ridx, cidx, uxv, uyv, uzv, dih, angv):
    cid = lax.axis_index("c")
    sid = lax.axis_index("s")
    wid = sid * NC + cid

    pltpu.sync_copy(
        (du_hbm.at[0].at[0], du_hbm.at[0].at[1], du_hbm.at[0].at[2],
         du_hbm.at[1].at[0], du_hbm.at[1].at[1], du_hbm.at[1].at[2]),
        (dx, dy, dz, tx, ty, tz),
    )

    @pl.loop(0, NPAD // L)
    def _add_loop(i):
      sl = pl.ds(i * L, L)
      dx[sl] = dx[sl] + tx[sl]
      dy[sl] = dy[sl] + ty[sl]
      dz[sl] = dz[sl] + tz[sl]

    if emit_nodes:
      ang_hbm, dsum_hbm = out_refs

      @pl.when(cid == 0)
      def _emit_nodes():
        base = sid * NSL

        @pl.loop(0, NSL // L)
        def _ang_loop(i):
          sl = pl.ds(i * L, L)
          vdx = dx[pl.ds(base + i * L, L)]
          vdy = dy[pl.ds(base + i * L, L)]
          vdz = dz[pl.ds(base + i * L, L)]
          angv[sl] = vdx * vdx + vdy * vdy + vdz * vdz

        nsl = pl.ds(base, NSL)
        pltpu.sync_copy(
            (angv, dx.at[nsl], dy.at[nsl], dz.at[nsl]),
            (ang_hbm.at[nsl], dsum_hbm.at[0].at[nsl],
             dsum_hbm.at[1].at[nsl], dsum_hbm.at[2].at[nsl]),
        )

    for ch in range(nchunk):
      lbase = wid * rpt + ch * bcrows
      rsl = pl.ds(row_lo + lbase, bcrows)
      pltpu.sync_copy(
          (row_hbm.at[rsl], col_hbm.at[rsl], u_hbm.at[0].at[rsl],
           u_hbm.at[1].at[rsl], u_hbm.at[2].at[rsl]),
          (ridx, cidx, uxv, uyv, uzv),
      )

      @pl.loop(0, bcrows)
      def _row_loop(j):
        for o in range(B // L):
          sl = pl.ds(o * L, L)
          r16 = ridx[j, sl]
          c16 = cidx[j, sl]
          kx = uxv[j, sl]
          ky = uyv[j, sl]
          kz = uzv[j, sl]
          vix = plsc.load_gather(dx, [r16])
          viy = plsc.load_gather(dy, [r16])
          viz = plsc.load_gather(dz, [r16])
          vjx = plsc.load_gather(dx, [c16])
          vjy = plsc.load_gather(dy, [c16])
          vjz = plsc.load_gather(dz, [c16])
          a = vix * kx + viy * ky + viz * kz
          b = vjx * kx + vjy * ky + vjz * kz
          ss = kx * kx + ky * ky + kz * kz
          dot = vix * vjx + viy * vjy + viz * vjz
          dih[j, sl] = dot - a * b * (jnp.float32(2.0) - ss)

      pltpu.sync_copy(dih, dih_hbm.at[pl.ds(lbase, bcrows)])

  return _pass_b


ROWS1 = NW * CROWS
ROWS2 = ROWS - ROWS1
_sc_pass_b1 = _make_sc_pass_b(0, CROWS, 1, True, CROWS)
_sc_pass_b2 = _make_sc_pass_b(ROWS1, 4 * CROWS, 2, False, 2 * CROWS)


def _tc_broadcast(xflat, nrows, bcr, w):
  g = nrows // (bcr * w)

  def body(x_ref, o_ref):
    x = x_ref[0]
    o_ref[...] = jnp.broadcast_to(
        x[:, :, None], (bcr, w, HIDDEN)
    ).reshape(bcr * w, HIDDEN)

  return pl.pallas_call(
      body,
      grid=(g,),
      in_specs=[pl.BlockSpec((1, bcr, w), lambda i: (i, 0, 0))],
      out_specs=pl.BlockSpec((bcr * w, HIDDEN), lambda i: (i, 0)),
      out_shape=jax.ShapeDtypeStruct((nrows, HIDDEN), jnp.float32),
  )(xflat.reshape(g, bcr, w))


_BCR = 50
_BW = _BCR * 128


def _tc_broadcast_part1(xflat):
  g = (ROWS1 * B) // _BW

  def body(x_ref, o_ref):
    x = x_ref[0]
    o_ref[...] = jnp.broadcast_to(
        x[:, :, None], (_BCR, 128, HIDDEN)
    ).reshape(_BW, HIDDEN)

  return pl.pallas_call(
      body,
      grid=(g,),
      in_specs=[pl.BlockSpec((1, _BCR, 128), lambda i: (i, 0, 0))],
      out_specs=pl.BlockSpec((_BW, HIDDEN), lambda i: (i, 0)),
      out_shape=jax.ShapeDtypeStruct((N_EDGES, HIDDEN), jnp.float32),
  )(xflat.reshape(g, _BCR, 128))


def _tc_broadcast_part2(xflat, part1):
  g = (ROWS2 * B) // _BW
  off = (ROWS1 * B) // _BW

  def body(x_ref, a_ref, o_ref):
    del a_ref
    x = x_ref[0]
    o_ref[...] = jnp.broadcast_to(
        x[:, :, None], (_BCR, 128, HIDDEN)
    ).reshape(_BW, HIDDEN)

  return pl.pallas_call(
      body,
      grid=(g,),
      in_specs=[
          pl.BlockSpec((1, _BCR, 128), lambda i: (i, 0, 0)),
          pl.BlockSpec(memory_space=pl.ANY),
      ],
      out_specs=pl.BlockSpec((_BW, HIDDEN), lambda i: (i + off, 0)),
      out_shape=jax.ShapeDtypeStruct((N_EDGES, HIDDEN), jnp.float32),
      input_output_aliases={1: 0},
  )(xflat.reshape(g, _BCR, 128), part1)


@jax.jit
def kernel(pos, edge_index, vector_features):
  del vector_features
  pos_t = jnp.zeros((3, NPAD), jnp.float32).at[:, :N_NODES].set(pos.T)
  row2d = edge_index[0].reshape(ROWS, B)
  col2d = edge_index[1].reshape(ROWS, B)

  u_hbm, du_part = _sc_pass_a(pos_t, row2d, col2d)
  dih1, ang, dsum = _sc_pass_b1(row2d, col2d, u_hbm, du_part)
  dih2 = _sc_pass_b2(row2d, col2d, u_hbm, du_part)
  if isinstance(dih2, (tuple, list)):
    dih2 = dih2[0]

  part1 = _tc_broadcast_part1(dih1.reshape(ROWS1 * B))
  angular_info = _tc_broadcast(ang[:N_NODES], N_NODES, 25, 80)
  dihedral_info = _tc_broadcast_part2(dih2.reshape(ROWS2 * B), part1)

  direction_units = dsum[:, :N_NODES].T
  return (angular_info, dihedral_info, direction_units)


# speedup vs baseline: 26.8643x; 1.0424x over previous
import functools

import jax
import jax.numpy as jnp
from jax import lax
from jax.experimental import pallas as pl
from jax.experimental.pallas import tpu as pltpu
from jax.experimental.pallas import tpu_sc as plsc

N_NODES = 10000
N_EDGES = 320000
HIDDEN = 128

NPAD = 10240
L = 16
NC = 2
NS = 16
NW = NC * NS
B = 80
ROWS = N_EDGES // B
RPT = ROWS // NW
CROWS = 25
NCHUNK = RPT // CROWS
NSL = NPAD // NS

_mesh = plsc.VectorSubcoreMesh(core_axis_name="c", subcore_axis_name="s")
_sc_params = pltpu.CompilerParams(use_tc_tiling_on_sc=False, needs_layout_passes=False)


def _rsqrt(s):
  i = plsc.bitcast(s, jnp.int32)
  i = jnp.int32(0x5F3759DF) - lax.shift_right_logical(i, 1)
  y = plsc.bitcast(i, jnp.float32)
  hs = s * jnp.float32(0.5)
  for _ in range(3):
    y = y * (jnp.float32(1.5) - hs * y * y)
  return y


@functools.partial(
    pl.kernel,
    out_type=(
        jax.ShapeDtypeStruct((3, ROWS, B), jnp.float32),
        jax.ShapeDtypeStruct((NC, 3, NPAD), jnp.float32),
    ),
    mesh=_mesh,
    compiler_params=_sc_params,
    scratch_types=dict(
        px=pltpu.VMEM((NPAD,), jnp.float32),
        py=pltpu.VMEM((NPAD,), jnp.float32),
        pz=pltpu.VMEM((NPAD,), jnp.float32),
        ridx=pltpu.VMEM((CROWS, B), jnp.int32),
        cidx=pltpu.VMEM((CROWS, B), jnp.int32),
        ux=pltpu.VMEM((CROWS, B), jnp.float32),
        uy=pltpu.VMEM((CROWS, B), jnp.float32),
        uz=pltpu.VMEM((CROWS, B), jnp.float32),
        nux=pltpu.VMEM((CROWS, B), jnp.float32),
        nuy=pltpu.VMEM((CROWS, B), jnp.float32),
        nuz=pltpu.VMEM((CROWS, B), jnp.float32),
        zb=pltpu.VMEM((NSL,), jnp.float32),
        accx=pltpu.VMEM_SHARED((NPAD,), jnp.float32),
        accy=pltpu.VMEM_SHARED((NPAD,), jnp.float32),
        accz=pltpu.VMEM_SHARED((NPAD,), jnp.float32),
        sem=pltpu.SemaphoreType.DMA,
    ),
)
def _sc_pass_a(pos_hbm, row_hbm, col_hbm, u_hbm, du_hbm, *, px, py, pz,
               ridx, cidx, ux, uy, uz, nux, nuy, nuz, zb, accx, accy, accz,
               sem):
  cid = lax.axis_index("c")
  sid = lax.axis_index("s")
  wid = sid * NC + cid

  pltpu.sync_copy(pos_hbm.at[0], px)
  pltpu.sync_copy(pos_hbm.at[1], py)
  pltpu.sync_copy(pos_hbm.at[2], pz)

  for i in range(NSL // L):
    zb[pl.ds(i * L, L)] = jnp.zeros((L,), jnp.float32)
  pltpu.sync_copy(zb, accx.at[pl.ds(sid * NSL, NSL)])
  pltpu.sync_copy(zb, accy.at[pl.ds(sid * NSL, NSL)])
  pltpu.sync_copy(zb, accz.at[pl.ds(sid * NSL, NSL)])
  plsc.subcore_barrier()

  def _scatter_row(j):
    pltpu.async_copy(ux.at[j], accx.at[ridx.at[j]], sem, add=True)
    pltpu.async_copy(uy.at[j], accy.at[ridx.at[j]], sem, add=True)
    pltpu.async_copy(uz.at[j], accz.at[ridx.at[j]], sem, add=True)
    pltpu.async_copy(nux.at[j], accx.at[cidx.at[j]], sem, add=True)
    pltpu.async_copy(nuy.at[j], accy.at[cidx.at[j]], sem, add=True)
    pltpu.async_copy(nuz.at[j], accz.at[cidx.at[j]], sem, add=True)

  def _drain_row(j):
    pltpu.make_async_copy(ux.at[j], accx.at[ridx.at[j]], sem).wait()
    pltpu.make_async_copy(uy.at[j], accy.at[ridx.at[j]], sem).wait()
    pltpu.make_async_copy(uz.at[j], accz.at[ridx.at[j]], sem).wait()
    pltpu.make_async_copy(nux.at[j], accx.at[cidx.at[j]], sem).wait()
    pltpu.make_async_copy(nuy.at[j], accy.at[cidx.at[j]], sem).wait()
    pltpu.make_async_copy(nuz.at[j], accz.at[cidx.at[j]], sem).wait()

  LAG = 5

  for ch in range(NCHUNK):
    rbase = wid * RPT + ch * CROWS
    pltpu.sync_copy(
        (row_hbm.at[pl.ds(rbase, CROWS)], col_hbm.at[pl.ds(rbase, CROWS)]),
        (ridx, cidx),
    )

    @pl.loop(0, CROWS)
    def _row_loop(j):
      for o in range(B // L):
        sl = pl.ds(o * L, L)
        r16 = ridx[j, sl]
        c16 = cidx[j, sl]
        ex = plsc.load_gather(px, [c16]) - plsc.load_gather(px, [r16])
        ey = plsc.load_gather(py, [c16]) - plsc.load_gather(py, [r16])
        ez = plsc.load_gather(pz, [c16]) - plsc.load_gather(pz, [r16])
        s = ex * ex + ey * ey + ez * ez
        norm = s * _rsqrt(s)
        inv = jnp.float32(1.0) / (norm + jnp.float32(1e-8))
        vx = ex * inv
        vy = ey * inv
        vz = ez * inv
        ux[j, sl] = vx
        uy[j, sl] = vy
        uz[j, sl] = vz
        nux[j, sl] = -vx
        nuy[j, sl] = -vy
        nuz[j, sl] = -vz
      _scatter_row(j)

      @pl.when(j >= LAG)
      def _():
        _drain_row(j - LAG)

    for r in range(LAG):
      _drain_row(CROWS - LAG + r)

    pltpu.sync_copy(
        (ux, uy, uz),
        (u_hbm.at[0].at[pl.ds(rbase, CROWS)],
         u_hbm.at[1].at[pl.ds(rbase, CROWS)],
         u_hbm.at[2].at[pl.ds(rbase, CROWS)]),
    )

  plsc.subcore_barrier()
  nsl = pl.ds(sid * NSL, NSL)
  pltpu.sync_copy(accx.at[nsl], du_hbm.at[cid].at[0].at[nsl])
  pltpu.sync_copy(accy.at[nsl], du_hbm.at[cid].at[1].at[nsl])
  pltpu.sync_copy(accz.at[nsl], du_hbm.at[cid].at[2].at[nsl])


def _make_sc_pass_b(row_lo, rpt, nchunk, emit_nodes, bcrows):
  assert rpt == nchunk * bcrows
  out_type = [jax.ShapeDtypeStruct((NW * rpt, B), jnp.float32)]
  if emit_nodes:
    out_type += [
        jax.ShapeDtypeStruct((NPAD,), jnp.float32),
        jax.ShapeDtypeStruct((3, NPAD), jnp.float32),
    ]

  @functools.partial(
      pl.kernel,
      out_type=tuple(out_type),
      mesh=_mesh,
      compiler_params=_sc_params,
      scratch_types=dict(
          dx=pltpu.VMEM((NPAD,), jnp.float32),
          dy=pltpu.VMEM((NPAD,), jnp.float32),
          dz=pltpu.VMEM((NPAD,), jnp.float32),
          tx=pltpu.VMEM((NPAD,), jnp.float32),
          ty=pltpu.VMEM((NPAD,), jnp.float32),
          tz=pltpu.VMEM((NPAD,), jnp.float32),
          ridx=pltpu.VMEM((bcrows, B), jnp.int32),
          cidx=pltpu.VMEM((bcrows, B), jnp.int32),
          uxv=pltpu.VMEM((bcrows, B), jnp.float32),
          uyv=pltpu.VMEM((bcrows, B), jnp.float32),
          uzv=pltpu.VMEM((bcrows, B), jnp.float32),
          dih=pltpu.VMEM((bcrows, B), jnp.float32),
          angv=pltpu.VMEM((NSL,), jnp.float32),
      ),
  )
  def _pass_b(row_hbm, col_hbm, u_hbm, du_hbm, dih_hbm, *out_refs,
              dx, dy, dz, tx, ty, tz, ridx, cidx, uxv, uyv, uzv, dih, angv):
    cid = lax.axis_index("c")
    sid = lax.axis_index("s")
    wid = sid * NC + cid

    pltpu.sync_copy(
        (du_hbm.at[0].at[0], du_hbm.at[0].at[1], du_hbm.at[0].at[2],
         du_hbm.at[1].at[0], du_hbm.at[1].at[1], du_hbm.at[1].at[2]),
        (dx, dy, dz, tx, ty, tz),
    )

    @pl.loop(0, NPAD // L)
    def _add_loop(i):
      sl = pl.ds(i * L, L)
      dx[sl] = dx[sl] + tx[sl]
      dy[sl] = dy[sl] + ty[sl]
      dz[sl] = dz[sl] + tz[sl]

    if emit_nodes:
      ang_hbm, dsum_hbm = out_refs

      @pl.when(cid == 0)
      def _emit_nodes():
        base = sid * NSL

        @pl.loop(0, NSL // L)
        def _ang_loop(i):
          sl = pl.ds(i * L, L)
          vdx = dx[pl.ds(base + i * L, L)]
          vdy = dy[pl.ds(base + i * L, L)]
          vdz = dz[pl.ds(base + i * L, L)]
          angv[sl] = vdx * vdx + vdy * vdy + vdz * vdz

        nsl = pl.ds(base, NSL)
        pltpu.sync_copy(
            (angv, dx.at[nsl], dy.at[nsl], dz.at[nsl]),
            (ang_hbm.at[nsl], dsum_hbm.at[0].at[nsl],
             dsum_hbm.at[1].at[nsl], dsum_hbm.at[2].at[nsl]),
        )

    for ch in range(nchunk):
      lbase = wid * rpt + ch * bcrows
      rsl = pl.ds(row_lo + lbase, bcrows)
      pltpu.sync_copy(
          (row_hbm.at[rsl], col_hbm.at[rsl], u_hbm.at[0].at[rsl],
           u_hbm.at[1].at[rsl], u_hbm.at[2].at[rsl]),
          (ridx, cidx, uxv, uyv, uzv),
      )

      @pl.loop(0, bcrows)
      def _row_loop(j):
        for o in range(B // L):
          sl = pl.ds(o * L, L)
          r16 = ridx[j, sl]
          c16 = cidx[j, sl]
          kx = uxv[j, sl]
          ky = uyv[j, sl]
          kz = uzv[j, sl]
          vix = plsc.load_gather(dx, [r16])
          viy = plsc.load_gather(dy, [r16])
          viz = plsc.load_gather(dz, [r16])
          vjx = plsc.load_gather(dx, [c16])
          vjy = plsc.load_gather(dy, [c16])
          vjz = plsc.load_gather(dz, [c16])
          a = vix * kx + viy * ky + viz * kz
          b = vjx * kx + vjy * ky + vjz * kz
          ss = kx * kx + ky * ky + kz * kz
          dot = vix * vjx + viy * vjy + viz * vjz
          dih[j, sl] = dot - a * b * (jnp.float32(2.0) - ss)

      pltpu.sync_copy(dih, dih_hbm.at[pl.ds(lbase, bcrows)])

  return _pass_b


RPT1 = 55
RPT2 = RPT - RPT1
ROWS1 = NW * RPT1
ROWS2 = ROWS - ROWS1
_sc_pass_b1 = _make_sc_pass_b(0, RPT1, 1, True, RPT1)
_sc_pass_b2 = _make_sc_pass_b(ROWS1, RPT2, 1, False, RPT2)


def _tc_broadcast(xflat, nrows, bcr, w):
  g = nrows // (bcr * w)

  def body(x_ref, o_ref):
    x = x_ref[0]
    o_ref[...] = jnp.broadcast_to(
        x[:, :, None], (bcr, w, HIDDEN)
    ).reshape(bcr * w, HIDDEN)

  return pl.pallas_call(
      body,
      grid=(g,),
      in_specs=[pl.BlockSpec((1, bcr, w), lambda i: (i, 0, 0))],
      out_specs=pl.BlockSpec((bcr * w, HIDDEN), lambda i: (i, 0)),
      out_shape=jax.ShapeDtypeStruct((nrows, HIDDEN), jnp.float32),
  )(xflat.reshape(g, bcr, w))


_BCR = 50
_BW = _BCR * 128


def _tc_broadcast_part1(xflat):
  g = (ROWS1 * B) // _BW

  def body(x_ref, o_ref):
    x = x_ref[0]
    o_ref[...] = jnp.broadcast_to(
        x[:, :, None], (_BCR, 128, HIDDEN)
    ).reshape(_BW, HIDDEN)

  return pl.pallas_call(
      body,
      grid=(g,),
      in_specs=[pl.BlockSpec((1, _BCR, 128), lambda i: (i, 0, 0))],
      out_specs=pl.BlockSpec((_BW, HIDDEN), lambda i: (i, 0)),
      out_shape=jax.ShapeDtypeStruct((N_EDGES, HIDDEN), jnp.float32),
  )(xflat.reshape(g, _BCR, 128))


def _tc_broadcast_part2(xflat, part1):
  g = (ROWS2 * B) // _BW
  off = (ROWS1 * B) // _BW

  def body(x_ref, a_ref, o_ref):
    del a_ref
    x = x_ref[0]
    o_ref[...] = jnp.broadcast_to(
        x[:, :, None], (_BCR, 128, HIDDEN)
    ).reshape(_BW, HIDDEN)

  return pl.pallas_call(
      body,
      grid=(g,),
      in_specs=[
          pl.BlockSpec((1, _BCR, 128), lambda i: (i, 0, 0)),
          pl.BlockSpec(memory_space=pl.ANY),
      ],
      out_specs=pl.BlockSpec((_BW, HIDDEN), lambda i: (i + off, 0)),
      out_shape=jax.ShapeDtypeStruct((N_EDGES, HIDDEN), jnp.float32),
      input_output_aliases={1: 0},
  )(xflat.reshape(g, _BCR, 128), part1)


@jax.jit
def kernel(pos, edge_index, vector_features):
  del vector_features
  pos_t = jnp.zeros((3, NPAD), jnp.float32).at[:, :N_NODES].set(pos.T)
  row2d = edge_index[0].reshape(ROWS, B)
  col2d = edge_index[1].reshape(ROWS, B)

  u_hbm, du_part = _sc_pass_a(pos_t, row2d, col2d)
  dih1, ang, dsum = _sc_pass_b1(row2d, col2d, u_hbm, du_part)
  dih2 = _sc_pass_b2(row2d, col2d, u_hbm, du_part)
  if isinstance(dih2, (tuple, list)):
    dih2 = dih2[0]

  part1 = _tc_broadcast_part1(dih1.reshape(ROWS1 * B))
  angular_info = _tc_broadcast(ang[:N_NODES], N_NODES, 25, 80)
  dihedral_info = _tc_broadcast_part2(dih2.reshape(ROWS2 * B), part1)

  direction_units = dsum[:, :N_NODES].T
  return (angular_info, dihedral_info, direction_units)
